# Initial kernel scaffold; baseline (speedup 1.0000x reference)
import jax
import jax.numpy as jnp
from jax.experimental import pallas as pl


def kernel(x, pos, edge_index, period_vec, batch, elem_embed, W_embed, b_embed, rbf_centers, W_msg, W_filter, W_self, W_attr, w_sh, b_conv, W_post0, b_post0, W_post1, b_post1, scale, shift):
    raise NotImplementedError("write your pallas kernel here")


# speedup vs baseline: 1.1351x; 1.1351x over previous
import functools

import jax
import jax.numpy as jnp
from jax import lax
from jax.experimental import pallas as pl
from jax.experimental.pallas import tpu as pltpu

GAMMA = 10.0
EPS = 1e-12
G_SEGMENTS = 64


def _silu(x):
    return x * jax.nn.sigmoid(x)


def _dsilu(x):
    s = jax.nn.sigmoid(x)
    return s * (1.0 + x * (1.0 - s))


def _blk(E):
    for eb in (2000, 1600, 1000, 800, 500, 400, 200, 100, 8):
        if E % eb == 0:
            return eb
    return E


def _t1_embed(xi, elem_pad, W_embed, b_embed, NB):
    NPAD = xi.shape[0]
    DA = elem_pad.shape[1]
    D = W_embed.shape[1]
    NE = elem_pad.shape[0]

    def body(xi_ref, elem_ref, we_ref, be_ref, xa_ref, h0_ref):
        ids = xi_ref[...]
        cols = lax.broadcasted_iota(jnp.int32, (NB, NE), 1)
        oh = (cols == ids).astype(jnp.float32)
        xa = jnp.dot(oh, elem_ref[...], preferred_element_type=jnp.float32)
        xa_ref[...] = xa
        h0_ref[...] = jnp.dot(xa, we_ref[...], preferred_element_type=jnp.float32) + be_ref[...]

    return pl.pallas_call(
        body,
        grid=(NPAD // NB,),
        in_specs=[
            pl.BlockSpec((NB, 1), lambda i: (i, 0)),
            pl.BlockSpec((NE, DA), lambda i: (0, 0)),
            pl.BlockSpec((DA, D), lambda i: (0, 0)),
            pl.BlockSpec((1, D), lambda i: (0, 0)),
        ],
        out_specs=[
            pl.BlockSpec((NB, DA), lambda i: (i, 0)),
            pl.BlockSpec((NB, D), lambda i: (i, 0)),
        ],
        out_shape=[
            jax.ShapeDtypeStruct((NPAD, DA), jnp.float32),
            jax.ShapeDtypeStruct((NPAD, D), jnp.float32),
        ],
    )(xi, elem_pad, W_embed, b_embed)


def _t_geom(evraw, w_sh, EB):
    E = evraw.shape[0]

    def body(ev_ref, wsh_ref, out_ref):
        ev = ev_ref[...]
        r = jnp.sqrt(jnp.sum(ev * ev, axis=1, keepdims=True) + EPS)
        u = ev[:, 0:3] / r
        wsh = wsh_ref[...]
        s_all = jnp.dot(u, wsh[:, 1:4].T, preferred_element_type=jnp.float32) + wsh[:, 0][None, :]
        z = jnp.zeros((ev.shape[0], 1), jnp.float32)
        out_ref[...] = jnp.concatenate([s_all, r, u, z], axis=1)

    return pl.pallas_call(
        body,
        grid=(E // EB,),
        in_specs=[
            pl.BlockSpec((EB, 8), lambda i: (i, 0)),
            pl.BlockSpec((3, 4), lambda i: (0, 0)),
        ],
        out_specs=pl.BlockSpec((EB, 8), lambda i: (i, 0)),
        out_shape=jax.ShapeDtypeStruct((E, 8), jnp.float32),
    )(evraw, w_sh)


def _t2_edge_fwd(hs, geom, Wm, Wf, centers, l, EB):
    E, D = hs.shape
    K = Wf.shape[0]

    def body(hs_ref, g_ref, wm_ref, wf_ref, c_ref, m_ref):
        geomb = g_ref[...]
        r = geomb[:, 3:4]
        s = geomb[:, l:l + 1]
        rbf = jnp.exp(-GAMMA * (r - c_ref[...]) ** 2)
        filt = jnp.dot(rbf, wf_ref[...], preferred_element_type=jnp.float32)
        gg = jnp.dot(hs_ref[...], wm_ref[...], preferred_element_type=jnp.float32)
        m_ref[...] = gg * filt * s

    return pl.pallas_call(
        body,
        grid=(E // EB,),
        in_specs=[
            pl.BlockSpec((EB, D), lambda i: (i, 0)),
            pl.BlockSpec((EB, 8), lambda i: (i, 0)),
            pl.BlockSpec((D, D), lambda i: (0, 0)),
            pl.BlockSpec((K, D), lambda i: (0, 0)),
            pl.BlockSpec((1, K), lambda i: (0, 0)),
        ],
        out_specs=pl.BlockSpec((EB, D), lambda i: (i, 0)),
        out_shape=jax.ShapeDtypeStruct((E, D), jnp.float32),
    )(hs, geom, Wm, Wf, centers)


def _t3_node(agg2, h, x_attr, Ws, Wa, bc, NB):
    NPAD, D = h.shape
    DA = x_attr.shape[1]

    def body(a_ref, h_ref, xa_ref, ws_ref, wa_ref, bc_ref, pre_ref, hn_ref):
        pre = (a_ref[0] + a_ref[1]
               + jnp.dot(h_ref[...], ws_ref[...], preferred_element_type=jnp.float32)
               + jnp.dot(xa_ref[...], wa_ref[...], preferred_element_type=jnp.float32)
               + bc_ref[...])
        pre_ref[...] = pre
        hn_ref[...] = _silu(pre)

    return pl.pallas_call(
        body,
        grid=(NPAD // NB,),
        in_specs=[
            pl.BlockSpec((2, NB, D), lambda i: (0, i, 0)),
            pl.BlockSpec((NB, D), lambda i: (i, 0)),
            pl.BlockSpec((NB, DA), lambda i: (i, 0)),
            pl.BlockSpec((D, D), lambda i: (0, 0)),
            pl.BlockSpec((DA, D), lambda i: (0, 0)),
            pl.BlockSpec((1, D), lambda i: (0, 0)),
        ],
        out_specs=[
            pl.BlockSpec((NB, D), lambda i: (i, 0)),
            pl.BlockSpec((NB, D), lambda i: (i, 0)),
        ],
        out_shape=[
            jax.ShapeDtypeStruct((NPAD, D), jnp.float32),
            jax.ShapeDtypeStruct((NPAD, D), jnp.float32),
        ],
    )(agg2, h, x_attr, Ws, Wa, bc)


def _t4_head(h, batch_p, W0, b0, W1, b1, scale, shift, NB):
    NPAD, D = h.shape
    DH = W0.shape[1]
    T = W1.shape[1]
    nblocks = NPAD // NB

    def body(h_ref, b_ref, w0_ref, b0_ref, w1_ref, b1_ref, sc_ref, sh_ref,
             z0_ref, en_ref):
        i = pl.program_id(0)
        z0 = jnp.dot(h_ref[...], w0_ref[...], preferred_element_type=jnp.float32) + b0_ref[...]
        z0_ref[...] = z0
        o = jnp.dot(_silu(z0), w1_ref[...], preferred_element_type=jnp.float32) + b1_ref[...]
        seg = b_ref[...]
        rows = lax.broadcasted_iota(jnp.int32, (G_SEGMENTS, NB), 0)
        oh = (rows == seg[:, 0][None, :]).astype(jnp.float32)
        part = jnp.dot(oh, o, preferred_element_type=jnp.float32)

        @pl.when(i == 0)
        def _():
            en_ref[...] = jnp.zeros_like(en_ref)

        en_ref[...] += part

        @pl.when(i == nblocks - 1)
        def _():
            en_ref[...] = en_ref[...] * sc_ref[...] + sh_ref[...]

    return pl.pallas_call(
        body,
        grid=(nblocks,),
        in_specs=[
            pl.BlockSpec((NB, D), lambda i: (i, 0)),
            pl.BlockSpec((NB, 1), lambda i: (i, 0)),
            pl.BlockSpec((D, DH), lambda i: (0, 0)),
            pl.BlockSpec((1, DH), lambda i: (0, 0)),
            pl.BlockSpec((DH, T), lambda i: (0, 0)),
            pl.BlockSpec((1, T), lambda i: (0, 0)),
            pl.BlockSpec((1, T), lambda i: (0, 0)),
            pl.BlockSpec((1, T), lambda i: (0, 0)),
        ],
        out_specs=[
            pl.BlockSpec((NB, DH), lambda i: (i, 0)),
            pl.BlockSpec((G_SEGMENTS, T), lambda i: (0, 0)),
        ],
        out_shape=[
            jax.ShapeDtypeStruct((NPAD, DH), jnp.float32),
            jax.ShapeDtypeStruct((G_SEGMENTS, T), jnp.float32),
        ],
    )(h, batch_p, W0, b0, W1, b1, scale, shift)


def _t5_head_bwd(z0, z1bar, W0T, NB):
    NPAD, DH = z0.shape
    D = W0T.shape[1]

    def body(z0_ref, zb_ref, w0t_ref, hb_ref):
        d = _dsilu(z0_ref[...])
        zb = zb_ref[...]
        h0 = jnp.dot(d * zb[0][None, :], w0t_ref[...], preferred_element_type=jnp.float32)
        h1 = jnp.dot(d * zb[1][None, :], w0t_ref[...], preferred_element_type=jnp.float32)
        hb_ref[...] = jnp.concatenate([h0, h1], axis=1)

    return pl.pallas_call(
        body,
        grid=(NPAD // NB,),
        in_specs=[
            pl.BlockSpec((NB, DH), lambda i: (i, 0)),
            pl.BlockSpec((2, DH), lambda i: (0, 0)),
            pl.BlockSpec((DH, D), lambda i: (0, 0)),
        ],
        out_specs=pl.BlockSpec((NB, 2 * D), lambda i: (i, 0)),
        out_shape=jax.ShapeDtypeStruct((NPAD, 2 * D), jnp.float32),
    )(z0, z1bar, W0T)


def _t6_node_bwd(hbar_parts, pre, WsT, NB, need_self):
    NPAD, D = pre.shape
    nparts = len(hbar_parts)

    def body(*refs):
        part_refs = refs[:nparts]
        pre_ref = refs[nparts]
        wst_ref = refs[nparts + 1]
        pb_ref = refs[nparts + 2]
        acc = part_refs[0][...]
        for p in part_refs[1:]:
            acc = acc + p[...]
        d = _dsilu(pre_ref[...])
        prebar = acc * jnp.concatenate([d, d], axis=1)
        pb_ref[...] = prebar
        if need_self:
            hs_ref = refs[nparts + 3]
            wst = wst_ref[...]
            h0 = jnp.dot(prebar[:, :D], wst, preferred_element_type=jnp.float32)
            h1 = jnp.dot(prebar[:, D:], wst, preferred_element_type=jnp.float32)
            hs_ref[...] = jnp.concatenate([h0, h1], axis=1)

    out_specs = [pl.BlockSpec((NB, 2 * D), lambda i: (i, 0))]
    out_shape = [jax.ShapeDtypeStruct((NPAD, 2 * D), jnp.float32)]
    if need_self:
        out_specs.append(pl.BlockSpec((NB, 2 * D), lambda i: (i, 0)))
        out_shape.append(jax.ShapeDtypeStruct((NPAD, 2 * D), jnp.float32))

    res = pl.pallas_call(
        body,
        grid=(NPAD // NB,),
        in_specs=[pl.BlockSpec((NB, 2 * D), lambda i: (i, 0)) for _ in range(nparts)]
        + [
            pl.BlockSpec((NB, D), lambda i: (i, 0)),
            pl.BlockSpec((D, D), lambda i: (0, 0)),
        ],
        out_specs=out_specs,
        out_shape=out_shape,
    )(*hbar_parts, pre, WsT)
    return res if need_self else (res[0], None)


def _t7_edge_bwd(mbar, hs, geom, gacc, Wm, Wf, WmT, WfT, wsh_l, centers, l, EB,
                 need_scatter):
    E, D = hs.shape
    K = Wf.shape[0]

    def body(mb_ref, hs_ref, g_ref, ga_ref, wm_ref, wf_ref, wmt_ref, wft_ref,
             wshl_ref, c_ref, *outs):
        geomb = g_ref[...]
        r = geomb[:, 3:4]
        s = geomb[:, l:l + 1]
        cen = c_ref[...]
        rbf = jnp.exp(-GAMMA * (r - cen) ** 2)
        drbf = (-2.0 * GAMMA) * (r - cen) * rbf
        filt = jnp.dot(rbf, wf_ref[...], preferred_element_type=jnp.float32)
        g = jnp.dot(hs_ref[...], wm_ref[...], preferred_element_type=jnp.float32)
        mb = mb_ref[...]
        wshl = wshl_ref[...]
        fs = filt * s
        acc_parts = []
        sc_parts = []
        for i in range(2):
            mbi = mb[:, i * D:(i + 1) * D]
            if need_scatter:
                gbar = mbi * fs
                sc_parts.append(jnp.dot(gbar, wmt_ref[...], preferred_element_type=jnp.float32))
            mg = mbi * g
            fbar = jnp.dot(mg * s, wft_ref[...], preferred_element_type=jnp.float32)
            rbar = jnp.sum(fbar * drbf, axis=1, keepdims=True)
            sb = jnp.sum(mg * filt, axis=1, keepdims=True)
            acc_parts.append(jnp.concatenate(
                [rbar, sb * wshl[0, 1], sb * wshl[0, 2], sb * wshl[0, 3]], axis=1))
        ga_out = outs[-1]
        ga_out[...] = ga_ref[...] + jnp.concatenate(acc_parts, axis=1)
        if need_scatter:
            outs[0][...] = jnp.concatenate(sc_parts, axis=1)

    out_specs = []
    out_shape = []
    if need_scatter:
        out_specs.append(pl.BlockSpec((EB, 2 * D), lambda i: (i, 0)))
        out_shape.append(jax.ShapeDtypeStruct((E, 2 * D), jnp.float32))
    out_specs.append(pl.BlockSpec((EB, 8), lambda i: (i, 0)))
    out_shape.append(jax.ShapeDtypeStruct((E, 8), jnp.float32))

    res = pl.pallas_call(
        body,
        grid=(E // EB,),
        in_specs=[
            pl.BlockSpec((EB, 2 * D), lambda i: (i, 0)),
            pl.BlockSpec((EB, D), lambda i: (i, 0)),
            pl.BlockSpec((EB, 8), lambda i: (i, 0)),
            pl.BlockSpec((EB, 8), lambda i: (i, 0)),
            pl.BlockSpec((D, D), lambda i: (0, 0)),
            pl.BlockSpec((K, D), lambda i: (0, 0)),
            pl.BlockSpec((D, D), lambda i: (0, 0)),
            pl.BlockSpec((D, K), lambda i: (0, 0)),
            pl.BlockSpec((1, 4), lambda i: (0, 0)),
            pl.BlockSpec((1, K), lambda i: (0, 0)),
        ],
        out_specs=out_specs,
        out_shape=out_shape,
    )(mbar, hs, geom, gacc, Wm, Wf, WmT, WfT, wsh_l, centers)
    if need_scatter:
        return res[0], res[1]
    return None, res[0]


def _t8_geom_bwd(geom, gacc, EB):
    E = geom.shape[0]

    def body(g_ref, ga_ref, out_ref):
        geomb = g_ref[...]
        ga = ga_ref[...]
        r = geomb[:, 3:4]
        u = geomb[:, 4:7]
        z = jnp.zeros((geomb.shape[0], 1), jnp.float32)
        parts = []
        for i in range(2):
            rb = ga[:, 4 * i:4 * i + 1]
            ub = ga[:, 4 * i + 1:4 * i + 4]
            uu = jnp.sum(ub * u, axis=1, keepdims=True)
            evb = (ub - u * uu) / r + u * rb
            parts.extend([evb, z])
        out_ref[...] = jnp.concatenate(parts, axis=1)

    return pl.pallas_call(
        body,
        grid=(E // EB,),
        in_specs=[
            pl.BlockSpec((EB, 8), lambda i: (i, 0)),
            pl.BlockSpec((EB, 8), lambda i: (i, 0)),
        ],
        out_specs=pl.BlockSpec((EB, 8), lambda i: (i, 0)),
        out_shape=jax.ShapeDtypeStruct((E, 8), jnp.float32),
    )(geom, gacc)


def _t9_forces(pb, NB):
    NPAD = pb.shape[1]

    def body(pb_ref, out_ref):
        out_ref[...] = -(pb_ref[0] + pb_ref[1])

    return pl.pallas_call(
        body,
        grid=(NPAD // NB,),
        in_specs=[pl.BlockSpec((2, NB, 8), lambda i: (0, i, 0))],
        out_specs=pl.BlockSpec((NB, 8), lambda i: (i, 0)),
        out_shape=jax.ShapeDtypeStruct((NPAD, 8), jnp.float32),
    )(pb)


def _sc_ev_build(pos_p, src, dst, period_vec):
    ev = pos_p[dst] - pos_p[src] + period_vec
    return jnp.concatenate([ev, jnp.zeros((ev.shape[0], 5), jnp.float32)], axis=1)


def _sc_gather(table, idx):
    return table[idx]


def _sc_scatter_add(data, idx, npad):
    part = jax.ops.segment_sum(data, idx, num_segments=npad)
    return jnp.stack([part, jnp.zeros_like(part)], axis=0)


def _sc_scatter_posbar(evb, src, dst, npad):
    p = jax.ops.segment_sum(evb, dst, num_segments=npad) - \
        jax.ops.segment_sum(evb, src, num_segments=npad)
    return jnp.stack([p, jnp.zeros_like(p)], axis=0)


def kernel(x, pos, edge_index, period_vec, batch, elem_embed, W_embed, b_embed,
           rbf_centers, W_msg, W_filter, W_self, W_attr, w_sh, b_conv,
           W_post0, b_post0, W_post1, b_post1, scale, shift):
    N = pos.shape[0]
    E = edge_index.shape[1]
    DA = elem_embed.shape[1]
    D = W_embed.shape[1]
    K = rbf_centers.shape[0]
    DH = W_post0.shape[1]
    T = W_post1.shape[1]

    NB = 1024
    NPAD = ((N + NB - 1) // NB) * NB
    EB = _blk(E)

    src = edge_index[0].astype(jnp.int32)
    dst = edge_index[1].astype(jnp.int32)
    xi_p = jnp.pad(x.reshape(-1, 1).astype(jnp.int32), ((0, NPAD - N), (0, 0)))
    batch_p = jnp.pad(batch.astype(jnp.int32).reshape(-1, 1),
                      ((0, NPAD - N), (0, 0)), constant_values=G_SEGMENTS)
    pos_p = jnp.pad(pos, ((0, NPAD - N), (0, 0)))

    NE_PAD = 128
    elem_pad = jnp.pad(elem_embed, ((0, NE_PAD - elem_embed.shape[0]), (0, 0)))
    centers = rbf_centers.reshape(1, K)
    b_embed2 = b_embed.reshape(1, D)
    b0_2 = b_post0.reshape(1, DH)
    b1_2 = b_post1.reshape(1, T)
    z1bar = scale[0][:, None] * W_post1.T
    W0T = W_post0.T
    WmT = [W_msg[l].T for l in range(3)]
    WfT = [W_filter[l].T for l in range(3)]
    WsT = [W_self[l].T for l in range(3)]

    x_attr, h0 = _t1_embed(xi_p, elem_pad, W_embed, b_embed2, NB)
    evraw = _sc_ev_build(pos_p, src, dst, period_vec)
    geom = _t_geom(evraw, w_sh, EB)

    hs_l = []
    pre_l = []
    h = h0
    for l in range(3):
        hs = _sc_gather(h, src)
        hs_l.append(hs)
        m = _t2_edge_fwd(hs, geom, W_msg[l], W_filter[l], centers, l, EB)
        agg2 = _sc_scatter_add(m, dst, NPAD)
        pre, h = _t3_node(agg2, h, x_attr, W_self[l], W_attr[l],
                          b_conv[l].reshape(1, D), NB)
        pre_l.append(pre)

    z0, energies = _t4_head(h, batch_p, W_post0, b0_2, W_post1, b1_2,
                            scale, shift, NB)

    hbar = _t5_head_bwd(z0, z1bar, W0T, NB)
    gacc = jnp.zeros((E, 8), jnp.float32)
    hbar_parts = [hbar]
    for l in range(2, -1, -1):
        prebar, hbar_self = _t6_node_bwd(hbar_parts, pre_l[l], WsT[l], NB,
                                         need_self=(l > 0))
        mbar = _sc_gather(prebar, dst)
        outsc, gacc = _t7_edge_bwd(mbar, hs_l[l], geom, gacc, W_msg[l],
                                   W_filter[l], WmT[l], WfT[l],
                                   w_sh[l].reshape(1, 4), centers, l, EB,
                                   need_scatter=(l > 0))
        if l > 0:
            hb2 = _sc_scatter_add(outsc, src, NPAD)
            hbar_parts = [hbar_self, hb2[0], hb2[1]]

    evb = _t8_geom_bwd(geom, gacc, EB)
    pb = _sc_scatter_posbar(evb, src, dst, NPAD)
    fb = _t9_forces(pb, NB)

    forces = jnp.stack([fb[:N, 0:3], fb[:N, 4:7]], axis=1)
    return (energies, forces)


# speedup vs baseline: 2.7467x; 2.4197x over previous
import functools

import jax
import jax.numpy as jnp
from jax import lax
from jax.experimental import pallas as pl
from jax.experimental.pallas import tpu as pltpu
from jax.experimental.pallas import tpu_sc as plsc

_NC, _NS = 2, 16
_NW = _NC * _NS
_CH = 80

GAMMA = 10.0
EPS = 1e-12
G_SEGMENTS = 64


def _silu(x):
    return x * jax.nn.sigmoid(x)


def _dsilu(x):
    s = jax.nn.sigmoid(x)
    return s * (1.0 + x * (1.0 - s))


def _blk(E):
    for eb in (2000, 1600, 1000, 800, 500, 400, 200, 100, 8):
        if E % eb == 0:
            return eb
    return E


def _t1_embed(xi, elem_pad, W_embed, b_embed, NB):
    NPAD = xi.shape[0]
    DA = elem_pad.shape[1]
    D = W_embed.shape[1]
    NE = elem_pad.shape[0]

    def body(xi_ref, elem_ref, we_ref, be_ref, xa_ref, h0_ref):
        ids = xi_ref[...]
        cols = lax.broadcasted_iota(jnp.int32, (NB, NE), 1)
        oh = (cols == ids).astype(jnp.float32)
        xa = jnp.dot(oh, elem_ref[...], preferred_element_type=jnp.float32)
        xa_ref[...] = xa
        h0_ref[...] = jnp.dot(xa, we_ref[...], preferred_element_type=jnp.float32) + be_ref[...]

    return pl.pallas_call(
        body,
        grid=(NPAD // NB,),
        in_specs=[
            pl.BlockSpec((NB, 1), lambda i: (i, 0)),
            pl.BlockSpec((NE, DA), lambda i: (0, 0)),
            pl.BlockSpec((DA, D), lambda i: (0, 0)),
            pl.BlockSpec((1, D), lambda i: (0, 0)),
        ],
        out_specs=[
            pl.BlockSpec((NB, DA), lambda i: (i, 0)),
            pl.BlockSpec((NB, D), lambda i: (i, 0)),
        ],
        out_shape=[
            jax.ShapeDtypeStruct((NPAD, DA), jnp.float32),
            jax.ShapeDtypeStruct((NPAD, D), jnp.float32),
        ],
    )(xi, elem_pad, W_embed, b_embed)


def _t_geom(pd16, ps16, pv, w_sh, EB):
    E = pd16.shape[0]

    def body(pd_ref, ps_ref, pv_ref, wsh_ref, out_ref):
        ev = pd_ref[...][:, 0:3] - ps_ref[...][:, 0:3] + pv_ref[...]
        r = jnp.sqrt(jnp.sum(ev * ev, axis=1, keepdims=True) + EPS)
        u = ev / r
        wsh = wsh_ref[...]
        s_all = jnp.dot(u, wsh[:, 1:4].T, preferred_element_type=jnp.float32) + wsh[:, 0][None, :]
        z = jnp.zeros((ev.shape[0], 1), jnp.float32)
        out_ref[...] = jnp.concatenate([s_all, r, u, z], axis=1)

    return pl.pallas_call(
        body,
        grid=(E // EB,),
        in_specs=[
            pl.BlockSpec((EB, 128), lambda i: (i, 0)),
            pl.BlockSpec((EB, 128), lambda i: (i, 0)),
            pl.BlockSpec((EB, 3), lambda i: (i, 0)),
            pl.BlockSpec((3, 4), lambda i: (0, 0)),
        ],
        out_specs=pl.BlockSpec((EB, 8), lambda i: (i, 0)),
        out_shape=jax.ShapeDtypeStruct((E, 8), jnp.float32),
    )(pd16, ps16, pv, w_sh)


def _t2_edge_fwd(hs, geom, Wm, Wf, centers, l, EB):
    E, D = hs.shape
    K = Wf.shape[0]

    def body(hs_ref, g_ref, wm_ref, wf_ref, c_ref, m_ref):
        geomb = g_ref[...]
        r = geomb[:, 3:4]
        s = geomb[:, l:l + 1]
        rbf = jnp.exp(-GAMMA * (r - c_ref[...]) ** 2)
        filt = jnp.dot(rbf, wf_ref[...], preferred_element_type=jnp.float32)
        gg = jnp.dot(hs_ref[...], wm_ref[...], preferred_element_type=jnp.float32)
        m_ref[...] = gg * filt * s

    return pl.pallas_call(
        body,
        grid=(E // EB,),
        in_specs=[
            pl.BlockSpec((EB, D), lambda i: (i, 0)),
            pl.BlockSpec((EB, 8), lambda i: (i, 0)),
            pl.BlockSpec((D, D), lambda i: (0, 0)),
            pl.BlockSpec((K, D), lambda i: (0, 0)),
            pl.BlockSpec((1, K), lambda i: (0, 0)),
        ],
        out_specs=pl.BlockSpec((EB, D), lambda i: (i, 0)),
        out_shape=jax.ShapeDtypeStruct((E, D), jnp.float32),
    )(hs, geom, Wm, Wf, centers)


def _t3_node(agg2, h, x_attr, Ws, Wa, bc, NB):
    NPAD, D = h.shape
    DA = x_attr.shape[1]

    def body(a_ref, h_ref, xa_ref, ws_ref, wa_ref, bc_ref, pre_ref, hn_ref):
        pre = (a_ref[0] + a_ref[1]
               + jnp.dot(h_ref[...], ws_ref[...], preferred_element_type=jnp.float32)
               + jnp.dot(xa_ref[...], wa_ref[...], preferred_element_type=jnp.float32)
               + bc_ref[...])
        pre_ref[...] = pre
        hn_ref[...] = _silu(pre)

    return pl.pallas_call(
        body,
        grid=(NPAD // NB,),
        in_specs=[
            pl.BlockSpec((2, NB, D), lambda i: (0, i, 0)),
            pl.BlockSpec((NB, D), lambda i: (i, 0)),
            pl.BlockSpec((NB, DA), lambda i: (i, 0)),
            pl.BlockSpec((D, D), lambda i: (0, 0)),
            pl.BlockSpec((DA, D), lambda i: (0, 0)),
            pl.BlockSpec((1, D), lambda i: (0, 0)),
        ],
        out_specs=[
            pl.BlockSpec((NB, D), lambda i: (i, 0)),
            pl.BlockSpec((NB, D), lambda i: (i, 0)),
        ],
        out_shape=[
            jax.ShapeDtypeStruct((NPAD, D), jnp.float32),
            jax.ShapeDtypeStruct((NPAD, D), jnp.float32),
        ],
    )(agg2, h, x_attr, Ws, Wa, bc)


def _t4_head(h, batch_p, W0, b0, W1, b1, scale, shift, NB):
    NPAD, D = h.shape
    DH = W0.shape[1]
    T = W1.shape[1]
    nblocks = NPAD // NB

    def body(h_ref, b_ref, w0_ref, b0_ref, w1_ref, b1_ref, sc_ref, sh_ref,
             z0_ref, en_ref):
        i = pl.program_id(0)
        z0 = jnp.dot(h_ref[...], w0_ref[...], preferred_element_type=jnp.float32) + b0_ref[...]
        z0_ref[...] = z0
        o = jnp.dot(_silu(z0), w1_ref[...], preferred_element_type=jnp.float32) + b1_ref[...]
        seg = b_ref[...]
        rows = lax.broadcasted_iota(jnp.int32, (G_SEGMENTS, NB), 0)
        oh = (rows == seg[:, 0][None, :]).astype(jnp.float32)
        part = jnp.dot(oh, o, preferred_element_type=jnp.float32)

        @pl.when(i == 0)
        def _():
            en_ref[...] = jnp.zeros_like(en_ref)

        en_ref[...] += part

        @pl.when(i == nblocks - 1)
        def _():
            en_ref[...] = en_ref[...] * sc_ref[...] + sh_ref[...]

    return pl.pallas_call(
        body,
        grid=(nblocks,),
        in_specs=[
            pl.BlockSpec((NB, D), lambda i: (i, 0)),
            pl.BlockSpec((NB, 1), lambda i: (i, 0)),
            pl.BlockSpec((D, DH), lambda i: (0, 0)),
            pl.BlockSpec((1, DH), lambda i: (0, 0)),
            pl.BlockSpec((DH, T), lambda i: (0, 0)),
            pl.BlockSpec((1, T), lambda i: (0, 0)),
            pl.BlockSpec((1, T), lambda i: (0, 0)),
            pl.BlockSpec((1, T), lambda i: (0, 0)),
        ],
        out_specs=[
            pl.BlockSpec((NB, DH), lambda i: (i, 0)),
            pl.BlockSpec((G_SEGMENTS, T), lambda i: (0, 0)),
        ],
        out_shape=[
            jax.ShapeDtypeStruct((NPAD, DH), jnp.float32),
            jax.ShapeDtypeStruct((G_SEGMENTS, T), jnp.float32),
        ],
    )(h, batch_p, W0, b0, W1, b1, scale, shift)


def _t5_head_bwd(z0, z1bar, W0T, NB):
    NPAD, DH = z0.shape
    D = W0T.shape[1]

    def body(z0_ref, zb_ref, w0t_ref, hb_ref):
        d = _dsilu(z0_ref[...])
        zb = zb_ref[...]
        h0 = jnp.dot(d * zb[0][None, :], w0t_ref[...], preferred_element_type=jnp.float32)
        h1 = jnp.dot(d * zb[1][None, :], w0t_ref[...], preferred_element_type=jnp.float32)
        hb_ref[...] = jnp.concatenate([h0, h1], axis=1)

    return pl.pallas_call(
        body,
        grid=(NPAD // NB,),
        in_specs=[
            pl.BlockSpec((NB, DH), lambda i: (i, 0)),
            pl.BlockSpec((2, DH), lambda i: (0, 0)),
            pl.BlockSpec((DH, D), lambda i: (0, 0)),
        ],
        out_specs=pl.BlockSpec((NB, 2 * D), lambda i: (i, 0)),
        out_shape=jax.ShapeDtypeStruct((NPAD, 2 * D), jnp.float32),
    )(z0, z1bar, W0T)


def _t6_node_bwd(hbar_full, hbar_halves, pre, WsT, NB, need_self):
    NPAD, D = pre.shape
    have_halves = hbar_halves is not None

    def body(*refs):
        i = 0
        hf_ref = refs[i]; i += 1
        if have_halves:
            pa_ref = refs[i]; pb_ref_in = refs[i + 1]; i += 2
        pre_ref = refs[i]; wst_ref = refs[i + 1]; i += 2
        pb_ref = refs[i]; i += 1
        acc = hf_ref[...]
        if have_halves:
            half0 = pa_ref[0] + pa_ref[1]
            half1 = pb_ref_in[0] + pb_ref_in[1]
            acc = acc + jnp.concatenate([half0, half1], axis=1)
        d = _dsilu(pre_ref[...])
        prebar = acc * jnp.concatenate([d, d], axis=1)
        pb_ref[...] = prebar
        if need_self:
            hs_ref = refs[i]
            wst = wst_ref[...]
            h0 = jnp.dot(prebar[:, :D], wst, preferred_element_type=jnp.float32)
            h1 = jnp.dot(prebar[:, D:], wst, preferred_element_type=jnp.float32)
            hs_ref[...] = jnp.concatenate([h0, h1], axis=1)

    in_specs = [pl.BlockSpec((NB, 2 * D), lambda i: (i, 0))]
    ins = [hbar_full]
    if have_halves:
        in_specs += [pl.BlockSpec((2, NB, D), lambda i: (0, i, 0))] * 2
        ins += [hbar_halves[0], hbar_halves[1]]
    in_specs += [
        pl.BlockSpec((NB, D), lambda i: (i, 0)),
        pl.BlockSpec((D, D), lambda i: (0, 0)),
    ]
    ins += [pre, WsT]

    out_specs = [pl.BlockSpec((NB, 2 * D), lambda i: (i, 0))]
    out_shape = [jax.ShapeDtypeStruct((NPAD, 2 * D), jnp.float32)]
    if need_self:
        out_specs.append(pl.BlockSpec((NB, 2 * D), lambda i: (i, 0)))
        out_shape.append(jax.ShapeDtypeStruct((NPAD, 2 * D), jnp.float32))

    res = pl.pallas_call(
        body,
        grid=(NPAD // NB,),
        in_specs=in_specs,
        out_specs=out_specs,
        out_shape=out_shape,
    )(*ins)
    return res if need_self else (res[0], None)


def _t7_edge_bwd(mbar, hs, geom, gacc, Wm, Wf, WmT, WfT, wsh_l, centers, l, EB,
                 need_scatter):
    E, D = hs.shape
    K = Wf.shape[0]

    def body(mb_ref, hs_ref, g_ref, ga_ref, wm_ref, wf_ref, wmt_ref, wft_ref,
             wshl_ref, c_ref, *outs):
        geomb = g_ref[...]
        r = geomb[:, 3:4]
        s = geomb[:, l:l + 1]
        cen = c_ref[...]
        rbf = jnp.exp(-GAMMA * (r - cen) ** 2)
        drbf = (-2.0 * GAMMA) * (r - cen) * rbf
        filt = jnp.dot(rbf, wf_ref[...], preferred_element_type=jnp.float32)
        g = jnp.dot(hs_ref[...], wm_ref[...], preferred_element_type=jnp.float32)
        mb = mb_ref[...]
        wshl = wshl_ref[...]
        fs = filt * s
        acc_parts = []
        sc_parts = []
        for i in range(2):
            mbi = mb[:, i * D:(i + 1) * D]
            if need_scatter:
                gbar = mbi * fs
                sc_parts.append(jnp.dot(gbar, wmt_ref[...], preferred_element_type=jnp.float32))
            mg = mbi * g
            fbar = jnp.dot(mg * s, wft_ref[...], preferred_element_type=jnp.float32)
            rbar = jnp.sum(fbar * drbf, axis=1, keepdims=True)
            sb = jnp.sum(mg * filt, axis=1, keepdims=True)
            acc_parts.append(jnp.concatenate(
                [rbar, sb * wshl[0, 1], sb * wshl[0, 2], sb * wshl[0, 3]], axis=1))
        ga_out = outs[-1]
        ga_out[...] = ga_ref[...] + jnp.concatenate(acc_parts, axis=1)
        if need_scatter:
            outs[0][...] = jnp.concatenate(sc_parts, axis=1)

    out_specs = []
    out_shape = []
    if need_scatter:
        out_specs.append(pl.BlockSpec((EB, 2 * D), lambda i: (i, 0)))
        out_shape.append(jax.ShapeDtypeStruct((E, 2 * D), jnp.float32))
    out_specs.append(pl.BlockSpec((EB, 8), lambda i: (i, 0)))
    out_shape.append(jax.ShapeDtypeStruct((E, 8), jnp.float32))

    res = pl.pallas_call(
        body,
        grid=(E // EB,),
        in_specs=[
            pl.BlockSpec((EB, 2 * D), lambda i: (i, 0)),
            pl.BlockSpec((EB, D), lambda i: (i, 0)),
            pl.BlockSpec((EB, 8), lambda i: (i, 0)),
            pl.BlockSpec((EB, 8), lambda i: (i, 0)),
            pl.BlockSpec((D, D), lambda i: (0, 0)),
            pl.BlockSpec((K, D), lambda i: (0, 0)),
            pl.BlockSpec((D, D), lambda i: (0, 0)),
            pl.BlockSpec((D, K), lambda i: (0, 0)),
            pl.BlockSpec((1, 4), lambda i: (0, 0)),
            pl.BlockSpec((1, K), lambda i: (0, 0)),
        ],
        out_specs=out_specs,
        out_shape=out_shape,
    )(mbar, hs, geom, gacc, Wm, Wf, WmT, WfT, wsh_l, centers)
    if need_scatter:
        return res[0], res[1]
    return None, res[0]


def _t8_geom_bwd(geom, gacc, EB):
    E = geom.shape[0]

    def body(g_ref, ga_ref, out_ref):
        geomb = g_ref[...]
        ga = ga_ref[...]
        r = geomb[:, 3:4]
        u = geomb[:, 4:7]
        z = jnp.zeros((geomb.shape[0], 1), jnp.float32)
        parts = []
        for i in range(2):
            rb = ga[:, 4 * i:4 * i + 1]
            ub = ga[:, 4 * i + 1:4 * i + 4]
            uu = jnp.sum(ub * u, axis=1, keepdims=True)
            evb = (ub - u * uu) / r + u * rb
            parts.extend([evb, z])
        ztail = jnp.zeros((geomb.shape[0], 120), jnp.float32)
        out_ref[...] = jnp.concatenate(parts + [ztail], axis=1)

    return pl.pallas_call(
        body,
        grid=(E // EB,),
        in_specs=[
            pl.BlockSpec((EB, 8), lambda i: (i, 0)),
            pl.BlockSpec((EB, 8), lambda i: (i, 0)),
        ],
        out_specs=pl.BlockSpec((EB, 128), lambda i: (i, 0)),
        out_shape=jax.ShapeDtypeStruct((E, 128), jnp.float32),
    )(geom, gacc)


def _t9_forces(pd, ps, NB):
    NPAD = pd.shape[1]

    def body(pd_ref, ps_ref, out_ref):
        f = (ps_ref[0] + ps_ref[1]) - (pd_ref[0] + pd_ref[1])
        out_ref[...] = f[:, 0:8]

    return pl.pallas_call(
        body,
        grid=(NPAD // NB,),
        in_specs=[pl.BlockSpec((2, NB, 128), lambda i: (0, i, 0))] * 2,
        out_specs=pl.BlockSpec((NB, 8), lambda i: (i, 0)),
        out_shape=jax.ShapeDtypeStruct((NPAD, 8), jnp.float32),
    )(pd, ps)


def _sc_gather(table, idx):
    C = table.shape[1]
    E = idx.shape[0]
    per_w = E // _NW
    nch = per_w // _CH
    npair = nch // 2
    mesh = plsc.VectorSubcoreMesh(core_axis_name="c", subcore_axis_name="s")

    @functools.partial(
        pl.kernel,
        out_type=jax.ShapeDtypeStruct((E, C), jnp.float32),
        mesh=mesh,
        scratch_types=[
            pltpu.VMEM((per_w,), jnp.int32),
            pltpu.VMEM((_CH, C), jnp.float32),
            pltpu.VMEM((_CH, C), jnp.float32),
            pltpu.SemaphoreType.DMA,
            pltpu.SemaphoreType.DMA,
        ],
    )
    def k(table_hbm, idx_hbm, out_hbm, idx_v, buf0, buf1, sem0, sem1):
        wid = lax.axis_index("s") * _NC + lax.axis_index("c")
        base = wid * per_w
        pltpu.sync_copy(idx_hbm.at[pl.ds(base, per_w)], idx_v)

        def step(i, carry):
            r0 = i * (2 * _CH)
            cp0 = pltpu.async_copy(table_hbm.at[idx_v.at[pl.ds(r0, _CH)]], buf0, sem0)
            cp1 = pltpu.async_copy(
                table_hbm.at[idx_v.at[pl.ds(r0 + _CH, _CH)]], buf1, sem1)
            cp0.wait()
            pltpu.sync_copy(buf0, out_hbm.at[pl.ds(base + r0, _CH)])
            cp1.wait()
            pltpu.sync_copy(buf1, out_hbm.at[pl.ds(base + r0 + _CH, _CH)])
            return carry

        lax.fori_loop(0, npair, step, 0)
        if nch % 2:
            r0 = (nch - 1) * _CH
            pltpu.async_copy(table_hbm.at[idx_v.at[pl.ds(r0, _CH)]], buf0, sem0).wait()
            pltpu.sync_copy(buf0, out_hbm.at[pl.ds(base + r0, _CH)])

    return k(table, idx)


def _sc_scatter_add(data, idx3, zeros_np, c0, C, npad):
    E = data.shape[0]
    per_w = E // _NW
    nch = per_w // _CH
    rows_t = npad // _NS
    mesh = plsc.VectorSubcoreMesh(core_axis_name="c", subcore_axis_name="s")

    @functools.partial(
        pl.kernel,
        out_type=jax.ShapeDtypeStruct((2, npad, C), jnp.float32),
        mesh=mesh,
        scratch_types=[
            pltpu.VMEM((nch, _CH), jnp.int32),
            pltpu.VMEM((_CH, C), jnp.float32),
            pltpu.VMEM((_CH, C), jnp.float32),
            pltpu.VMEM_SHARED((npad, C), jnp.float32),
            pltpu.SemaphoreType.DMA,
            pltpu.SemaphoreType.DMA,
        ],
    )
    def k(data_hbm, idx_hbm, z_hbm, out_hbm, idx_v, dbuf0, dbuf1, accum,
          sem0, sem1):
        c_ax = lax.axis_index("c")
        s_ax = lax.axis_index("s")
        wid = s_ax * _NC + c_ax
        base = wid * per_w
        row0 = s_ax * rows_t
        pltpu.sync_copy(z_hbm.at[pl.ds(row0, rows_t)],
                        accum.at[pl.ds(row0, rows_t)])
        pltpu.sync_copy(idx_hbm.at[wid], idx_v)
        plsc.subcore_barrier()

        def step(i, carry):
            j0 = i * 2
            cp0 = pltpu.async_copy(
                data_hbm.at[pl.ds(base + j0 * _CH, _CH), pl.ds(c0, C)],
                dbuf0, sem0)
            cp1 = pltpu.async_copy(
                data_hbm.at[pl.ds(base + (j0 + 1) * _CH, _CH), pl.ds(c0, C)],
                dbuf1, sem1)
            cp0.wait()
            pltpu.sync_copy(dbuf0, accum.at[idx_v.at[j0]], add=True)
            cp1.wait()
            pltpu.sync_copy(dbuf1, accum.at[idx_v.at[j0 + 1]], add=True)
            return carry

        lax.fori_loop(0, nch // 2, step, 0)
        if nch % 2:
            j = nch - 1
            pltpu.async_copy(
                data_hbm.at[pl.ds(base + j * _CH, _CH), pl.ds(c0, C)],
                dbuf0, sem0).wait()
            pltpu.sync_copy(dbuf0, accum.at[idx_v.at[j]], add=True)
        plsc.subcore_barrier()
        pltpu.sync_copy(accum.at[pl.ds(row0, rows_t)],
                        out_hbm.at[c_ax, pl.ds(row0, rows_t)])

    return k(data, idx3, zeros_np)


def kernel(x, pos, edge_index, period_vec, batch, elem_embed, W_embed, b_embed,
           rbf_centers, W_msg, W_filter, W_self, W_attr, w_sh, b_conv,
           W_post0, b_post0, W_post1, b_post1, scale, shift):
    N = pos.shape[0]
    E = edge_index.shape[1]
    DA = elem_embed.shape[1]
    D = W_embed.shape[1]
    K = rbf_centers.shape[0]
    DH = W_post0.shape[1]
    T = W_post1.shape[1]

    NB = 1024
    NPAD = ((N + NB - 1) // NB) * NB
    EB = _blk(E)

    src = edge_index[0].astype(jnp.int32)
    dst = edge_index[1].astype(jnp.int32)
    xi_p = jnp.pad(x.reshape(-1, 1).astype(jnp.int32), ((0, NPAD - N), (0, 0)))
    batch_p = jnp.pad(batch.astype(jnp.int32).reshape(-1, 1),
                      ((0, NPAD - N), (0, 0)), constant_values=G_SEGMENTS)
    pos128 = jnp.pad(pos, ((0, NPAD - N), (0, 125)))
    per_w = E // _NW
    nch = per_w // _CH
    src3 = src.reshape(_NW, nch, _CH)
    dst3 = dst.reshape(_NW, nch, _CH)
    z128 = jnp.zeros((NPAD, D), jnp.float32)

    NE_PAD = 128
    elem_pad = jnp.pad(elem_embed, ((0, NE_PAD - elem_embed.shape[0]), (0, 0)))
    centers = rbf_centers.reshape(1, K)
    b_embed2 = b_embed.reshape(1, D)
    b0_2 = b_post0.reshape(1, DH)
    b1_2 = b_post1.reshape(1, T)
    z1bar = scale[0][:, None] * W_post1.T
    W0T = W_post0.T
    WmT = [W_msg[l].T for l in range(3)]
    WfT = [W_filter[l].T for l in range(3)]
    WsT = [W_self[l].T for l in range(3)]

    x_attr, h0 = _t1_embed(xi_p, elem_pad, W_embed, b_embed2, NB)
    pd16 = _sc_gather(pos128, dst)
    ps16 = _sc_gather(pos128, src)
    geom = _t_geom(pd16, ps16, period_vec, w_sh, EB)

    hs_l = []
    pre_l = []
    h = h0
    for l in range(3):
        hs = _sc_gather(h, src)
        hs_l.append(hs)
        m = _t2_edge_fwd(hs, geom, W_msg[l], W_filter[l], centers, l, EB)
        agg2 = _sc_scatter_add(m, dst3, z128, 0, D, NPAD)
        pre, h = _t3_node(agg2, h, x_attr, W_self[l], W_attr[l],
                          b_conv[l].reshape(1, D), NB)
        pre_l.append(pre)

    z0, energies = _t4_head(h, batch_p, W_post0, b0_2, W_post1, b1_2,
                            scale, shift, NB)

    hbar = _t5_head_bwd(z0, z1bar, W0T, NB)
    gacc = jnp.zeros((E, 8), jnp.float32)
    hbar_halves = None
    for l in range(2, -1, -1):
        prebar, hbar_self = _t6_node_bwd(hbar, hbar_halves, pre_l[l], WsT[l],
                                         NB, need_self=(l > 0))
        mbar = _sc_gather(prebar, dst)
        outsc, gacc = _t7_edge_bwd(mbar, hs_l[l], geom, gacc, W_msg[l],
                                   W_filter[l], WmT[l], WfT[l],
                                   w_sh[l].reshape(1, 4), centers, l, EB,
                                   need_scatter=(l > 0))
        if l > 0:
            hbar = hbar_self
            hbar_halves = (_sc_scatter_add(outsc, src3, z128, 0, D, NPAD),
                           _sc_scatter_add(outsc, src3, z128, D, D, NPAD))

    evb = _t8_geom_bwd(geom, gacc, EB)
    pd = _sc_scatter_add(evb, dst3, z128, 0, D, NPAD)
    psc = _sc_scatter_add(evb, src3, z128, 0, D, NPAD)
    fb = _t9_forces(pd, psc, NB)

    forces = jnp.stack([fb[:N, 0:3], fb[:N, 4:7]], axis=1)
    return (energies, forces)


# speedup vs baseline: 2.7791x; 1.0118x over previous
import functools

import jax
import jax.numpy as jnp
from jax import lax
from jax.experimental import pallas as pl
from jax.experimental.pallas import tpu as pltpu
from jax.experimental.pallas import tpu_sc as plsc

_NC, _NS = 2, 16
_NW = _NC * _NS
_CH = 80

GAMMA = 10.0
EPS = 1e-12
G_SEGMENTS = 64


def _silu(x):
    return x * jax.nn.sigmoid(x)


def _dsilu(x):
    s = jax.nn.sigmoid(x)
    return s * (1.0 + x * (1.0 - s))


def _blk(E):
    for eb in (2000, 1600, 1000, 800, 500, 400, 200, 100, 8):
        if E % eb == 0:
            return eb
    return E


def _t1_embed(xi, elem_pad, W_embed, b_embed, NB):
    NPAD = xi.shape[0]
    DA = elem_pad.shape[1]
    D = W_embed.shape[1]
    NE = elem_pad.shape[0]

    def body(xi_ref, elem_ref, we_ref, be_ref, xa_ref, h0_ref):
        ids = xi_ref[...]
        cols = lax.broadcasted_iota(jnp.int32, (NB, NE), 1)
        oh = (cols == ids).astype(jnp.float32)
        xa = jnp.dot(oh, elem_ref[...], preferred_element_type=jnp.float32)
        xa_ref[...] = xa
        h0_ref[...] = jnp.dot(xa, we_ref[...], preferred_element_type=jnp.float32) + be_ref[...]

    return pl.pallas_call(
        body,
        grid=(NPAD // NB,),
        in_specs=[
            pl.BlockSpec((NB, 1), lambda i: (i, 0)),
            pl.BlockSpec((NE, DA), lambda i: (0, 0)),
            pl.BlockSpec((DA, D), lambda i: (0, 0)),
            pl.BlockSpec((1, D), lambda i: (0, 0)),
        ],
        out_specs=[
            pl.BlockSpec((NB, DA), lambda i: (i, 0)),
            pl.BlockSpec((NB, D), lambda i: (i, 0)),
        ],
        out_shape=[
            jax.ShapeDtypeStruct((NPAD, DA), jnp.float32),
            jax.ShapeDtypeStruct((NPAD, D), jnp.float32),
        ],
    )(xi, elem_pad, W_embed, b_embed)


def _t_geom(pg, pv, w_sh, EB):
    E = pg.shape[1]

    def body(pd_ref, ps_ref, pv_ref, wsh_ref, out_ref):
        ev = pd_ref[0][:, 0:3] - ps_ref[0][:, 0:3] + pv_ref[...]
        r = jnp.sqrt(jnp.sum(ev * ev, axis=1, keepdims=True) + EPS)
        u = ev / r
        wsh = wsh_ref[...]
        s_all = jnp.dot(u, wsh[:, 1:4].T, preferred_element_type=jnp.float32) + wsh[:, 0][None, :]
        z = jnp.zeros((ev.shape[0], 1), jnp.float32)
        out_ref[...] = jnp.concatenate([s_all, r, u, z], axis=1)

    return pl.pallas_call(
        body,
        grid=(E // EB,),
        in_specs=[
            pl.BlockSpec((1, EB, 128), lambda i: (0, i, 0)),
            pl.BlockSpec((1, EB, 128), lambda i: (1, i, 0)),
            pl.BlockSpec((EB, 3), lambda i: (i, 0)),
            pl.BlockSpec((3, 4), lambda i: (0, 0)),
        ],
        out_specs=pl.BlockSpec((EB, 8), lambda i: (i, 0)),
        out_shape=jax.ShapeDtypeStruct((E, 8), jnp.float32),
    )(pg, pg, pv, w_sh)


def _t2_edge_fwd(hs, geom, Wm, Wf, centers, l, EB):
    E, D = hs.shape
    K = Wf.shape[0]

    def body(hs_ref, g_ref, wm_ref, wf_ref, c_ref, m_ref):
        geomb = g_ref[...]
        r = geomb[:, 3:4]
        s = geomb[:, l:l + 1]
        rbf = jnp.exp(-GAMMA * (r - c_ref[...]) ** 2)
        filt = jnp.dot(rbf, wf_ref[...], preferred_element_type=jnp.float32)
        gg = jnp.dot(hs_ref[...], wm_ref[...], preferred_element_type=jnp.float32)
        m_ref[...] = gg * filt * s

    return pl.pallas_call(
        body,
        grid=(E // EB,),
        in_specs=[
            pl.BlockSpec((EB, D), lambda i: (i, 0)),
            pl.BlockSpec((EB, 8), lambda i: (i, 0)),
            pl.BlockSpec((D, D), lambda i: (0, 0)),
            pl.BlockSpec((K, D), lambda i: (0, 0)),
            pl.BlockSpec((1, K), lambda i: (0, 0)),
        ],
        out_specs=pl.BlockSpec((EB, D), lambda i: (i, 0)),
        out_shape=jax.ShapeDtypeStruct((E, D), jnp.float32),
    )(hs, geom, Wm, Wf, centers)


def _t3_node(agg2, h, x_attr, Ws, Wa, bc, NB):
    NPAD, D = h.shape
    DA = x_attr.shape[1]

    def body(a_ref, h_ref, xa_ref, ws_ref, wa_ref, bc_ref, pre_ref, hn_ref):
        pre = (a_ref[0] + a_ref[1]
               + jnp.dot(h_ref[...], ws_ref[...], preferred_element_type=jnp.float32)
               + jnp.dot(xa_ref[...], wa_ref[...], preferred_element_type=jnp.float32)
               + bc_ref[...])
        pre_ref[...] = pre
        hn_ref[...] = _silu(pre)

    return pl.pallas_call(
        body,
        grid=(NPAD // NB,),
        in_specs=[
            pl.BlockSpec((2, NB, D), lambda i: (0, i, 0)),
            pl.BlockSpec((NB, D), lambda i: (i, 0)),
            pl.BlockSpec((NB, DA), lambda i: (i, 0)),
            pl.BlockSpec((D, D), lambda i: (0, 0)),
            pl.BlockSpec((DA, D), lambda i: (0, 0)),
            pl.BlockSpec((1, D), lambda i: (0, 0)),
        ],
        out_specs=[
            pl.BlockSpec((NB, D), lambda i: (i, 0)),
            pl.BlockSpec((NB, D), lambda i: (i, 0)),
        ],
        out_shape=[
            jax.ShapeDtypeStruct((NPAD, D), jnp.float32),
            jax.ShapeDtypeStruct((NPAD, D), jnp.float32),
        ],
    )(agg2, h, x_attr, Ws, Wa, bc)


def _t4_head(h, batch_p, W0, b0, W1, b1, scale, shift, NB):
    NPAD, D = h.shape
    DH = W0.shape[1]
    T = W1.shape[1]
    nblocks = NPAD // NB

    def body(h_ref, b_ref, w0_ref, b0_ref, w1_ref, b1_ref, sc_ref, sh_ref,
             z0_ref, en_ref):
        i = pl.program_id(0)
        z0 = jnp.dot(h_ref[...], w0_ref[...], preferred_element_type=jnp.float32) + b0_ref[...]
        z0_ref[...] = z0
        o = jnp.dot(_silu(z0), w1_ref[...], preferred_element_type=jnp.float32) + b1_ref[...]
        seg = b_ref[...]
        rows = lax.broadcasted_iota(jnp.int32, (G_SEGMENTS, NB), 0)
        oh = (rows == seg[:, 0][None, :]).astype(jnp.float32)
        part = jnp.dot(oh, o, preferred_element_type=jnp.float32)

        @pl.when(i == 0)
        def _():
            en_ref[...] = jnp.zeros_like(en_ref)

        en_ref[...] += part

        @pl.when(i == nblocks - 1)
        def _():
            en_ref[...] = en_ref[...] * sc_ref[...] + sh_ref[...]

    return pl.pallas_call(
        body,
        grid=(nblocks,),
        in_specs=[
            pl.BlockSpec((NB, D), lambda i: (i, 0)),
            pl.BlockSpec((NB, 1), lambda i: (i, 0)),
            pl.BlockSpec((D, DH), lambda i: (0, 0)),
            pl.BlockSpec((1, DH), lambda i: (0, 0)),
            pl.BlockSpec((DH, T), lambda i: (0, 0)),
            pl.BlockSpec((1, T), lambda i: (0, 0)),
            pl.BlockSpec((1, T), lambda i: (0, 0)),
            pl.BlockSpec((1, T), lambda i: (0, 0)),
        ],
        out_specs=[
            pl.BlockSpec((NB, DH), lambda i: (i, 0)),
            pl.BlockSpec((G_SEGMENTS, T), lambda i: (0, 0)),
        ],
        out_shape=[
            jax.ShapeDtypeStruct((NPAD, DH), jnp.float32),
            jax.ShapeDtypeStruct((G_SEGMENTS, T), jnp.float32),
        ],
    )(h, batch_p, W0, b0, W1, b1, scale, shift)


def _t5_head_bwd(z0, z1bar, W0T, NB):
    NPAD, DH = z0.shape
    D = W0T.shape[1]

    def body(z0_ref, zb_ref, w0t_ref, hb_ref):
        d = _dsilu(z0_ref[...])
        zb = zb_ref[...]
        h0 = jnp.dot(d * zb[0][None, :], w0t_ref[...], preferred_element_type=jnp.float32)
        h1 = jnp.dot(d * zb[1][None, :], w0t_ref[...], preferred_element_type=jnp.float32)
        hb_ref[...] = jnp.concatenate([h0, h1], axis=1)

    return pl.pallas_call(
        body,
        grid=(NPAD // NB,),
        in_specs=[
            pl.BlockSpec((NB, DH), lambda i: (i, 0)),
            pl.BlockSpec((2, DH), lambda i: (0, 0)),
            pl.BlockSpec((DH, D), lambda i: (0, 0)),
        ],
        out_specs=pl.BlockSpec((NB, 2 * D), lambda i: (i, 0)),
        out_shape=jax.ShapeDtypeStruct((NPAD, 2 * D), jnp.float32),
    )(z0, z1bar, W0T)


def _t6_node_bwd(hbar_full, hbar_halves, pre, WsT, NB, need_self):
    NPAD, D = pre.shape
    have_halves = hbar_halves is not None

    def body(*refs):
        i = 0
        hf_ref = refs[i]; i += 1
        if have_halves:
            ha_ref = refs[i]; i += 1
        pre_ref = refs[i]; wst_ref = refs[i + 1]; i += 2
        pb_ref = refs[i]; i += 1
        acc = hf_ref[...]
        if have_halves:
            acc = acc + jnp.concatenate([ha_ref[0], ha_ref[1]], axis=1)
        d = _dsilu(pre_ref[...])
        prebar = acc * jnp.concatenate([d, d], axis=1)
        pb_ref[...] = prebar
        if need_self:
            hs_ref = refs[i]
            wst = wst_ref[...]
            h0 = jnp.dot(prebar[:, :D], wst, preferred_element_type=jnp.float32)
            h1 = jnp.dot(prebar[:, D:], wst, preferred_element_type=jnp.float32)
            hs_ref[...] = jnp.concatenate([h0, h1], axis=1)

    in_specs = [pl.BlockSpec((NB, 2 * D), lambda i: (i, 0))]
    ins = [hbar_full]
    if have_halves:
        in_specs += [pl.BlockSpec((2, NB, D), lambda i: (0, i, 0))]
        ins += [hbar_halves]
    in_specs += [
        pl.BlockSpec((NB, D), lambda i: (i, 0)),
        pl.BlockSpec((D, D), lambda i: (0, 0)),
    ]
    ins += [pre, WsT]

    out_specs = [pl.BlockSpec((NB, 2 * D), lambda i: (i, 0))]
    out_shape = [jax.ShapeDtypeStruct((NPAD, 2 * D), jnp.float32)]
    if need_self:
        out_specs.append(pl.BlockSpec((NB, 2 * D), lambda i: (i, 0)))
        out_shape.append(jax.ShapeDtypeStruct((NPAD, 2 * D), jnp.float32))

    res = pl.pallas_call(
        body,
        grid=(NPAD // NB,),
        in_specs=in_specs,
        out_specs=out_specs,
        out_shape=out_shape,
    )(*ins)
    return res if need_self else (res[0], None)


def _t7_edge_bwd(mbar, hs, geom, gacc, Wm, Wf, WmT, WfT, wsh_l, centers, l, EB,
                 need_scatter):
    E, D = hs.shape
    K = Wf.shape[0]

    def body(mb_ref, hs_ref, g_ref, ga_ref, wm_ref, wf_ref, wmt_ref, wft_ref,
             wshl_ref, c_ref, *outs):
        geomb = g_ref[...]
        r = geomb[:, 3:4]
        s = geomb[:, l:l + 1]
        cen = c_ref[...]
        rbf = jnp.exp(-GAMMA * (r - cen) ** 2)
        drbf = (-2.0 * GAMMA) * (r - cen) * rbf
        filt = jnp.dot(rbf, wf_ref[...], preferred_element_type=jnp.float32)
        g = jnp.dot(hs_ref[...], wm_ref[...], preferred_element_type=jnp.float32)
        mb = mb_ref[...]
        wshl = wshl_ref[...]
        fs = filt * s
        acc_parts = []
        sc_parts = []
        for i in range(2):
            mbi = mb[:, i * D:(i + 1) * D]
            if need_scatter:
                gbar = mbi * fs
                sc_parts.append(jnp.dot(gbar, wmt_ref[...], preferred_element_type=jnp.float32))
            mg = mbi * g
            fbar = jnp.dot(mg * s, wft_ref[...], preferred_element_type=jnp.float32)
            rbar = jnp.sum(fbar * drbf, axis=1, keepdims=True)
            sb = jnp.sum(mg * filt, axis=1, keepdims=True)
            acc_parts.append(jnp.concatenate(
                [rbar, sb * wshl[0, 1], sb * wshl[0, 2], sb * wshl[0, 3]], axis=1))
        ga_out = outs[-1]
        ga_out[...] = ga_ref[...] + jnp.concatenate(acc_parts, axis=1)
        if need_scatter:
            outs[0][...] = jnp.concatenate(sc_parts, axis=1)

    out_specs = []
    out_shape = []
    if need_scatter:
        out_specs.append(pl.BlockSpec((EB, 2 * D), lambda i: (i, 0)))
        out_shape.append(jax.ShapeDtypeStruct((E, 2 * D), jnp.float32))
    out_specs.append(pl.BlockSpec((EB, 8), lambda i: (i, 0)))
    out_shape.append(jax.ShapeDtypeStruct((E, 8), jnp.float32))

    res = pl.pallas_call(
        body,
        grid=(E // EB,),
        in_specs=[
            pl.BlockSpec((EB, 2 * D), lambda i: (i, 0)),
            pl.BlockSpec((EB, D), lambda i: (i, 0)),
            pl.BlockSpec((EB, 8), lambda i: (i, 0)),
            pl.BlockSpec((EB, 8), lambda i: (i, 0)),
            pl.BlockSpec((D, D), lambda i: (0, 0)),
            pl.BlockSpec((K, D), lambda i: (0, 0)),
            pl.BlockSpec((D, D), lambda i: (0, 0)),
            pl.BlockSpec((D, K), lambda i: (0, 0)),
            pl.BlockSpec((1, 4), lambda i: (0, 0)),
            pl.BlockSpec((1, K), lambda i: (0, 0)),
        ],
        out_specs=out_specs,
        out_shape=out_shape,
    )(mbar, hs, geom, gacc, Wm, Wf, WmT, WfT, wsh_l, centers)
    if need_scatter:
        return res[0], res[1]
    return None, res[0]


def _t8_geom_bwd(geom, gacc, EB):
    E = geom.shape[0]

    def body(g_ref, ga_ref, out_ref):
        geomb = g_ref[...]
        ga = ga_ref[...]
        r = geomb[:, 3:4]
        u = geomb[:, 4:7]
        z = jnp.zeros((geomb.shape[0], 1), jnp.float32)
        parts = []
        for i in range(2):
            rb = ga[:, 4 * i:4 * i + 1]
            ub = ga[:, 4 * i + 1:4 * i + 4]
            uu = jnp.sum(ub * u, axis=1, keepdims=True)
            evb = (ub - u * uu) / r + u * rb
            parts.extend([evb, z])
        ztail = jnp.zeros((geomb.shape[0], 120), jnp.float32)
        out_ref[...] = jnp.concatenate(parts + [ztail], axis=1)

    return pl.pallas_call(
        body,
        grid=(E // EB,),
        in_specs=[
            pl.BlockSpec((EB, 8), lambda i: (i, 0)),
            pl.BlockSpec((EB, 8), lambda i: (i, 0)),
        ],
        out_specs=pl.BlockSpec((EB, 128), lambda i: (i, 0)),
        out_shape=jax.ShapeDtypeStruct((E, 128), jnp.float32),
    )(geom, gacc)


def _t9_forces(pb, NB):
    NPAD = pb.shape[1]

    def body(pb_ref, out_ref):
        f = pb_ref[1] - pb_ref[0]
        out_ref[...] = f[:, 0:8]

    return pl.pallas_call(
        body,
        grid=(NPAD // NB,),
        in_specs=[pl.BlockSpec((2, NB, 128), lambda i: (0, i, 0))],
        out_specs=pl.BlockSpec((NB, 8), lambda i: (i, 0)),
        out_shape=jax.ShapeDtypeStruct((NPAD, 8), jnp.float32),
    )(pb)


def _sc_gather(table, idx):
    C = table.shape[1]
    E = idx.shape[0]
    per_w = E // _NW
    nch = per_w // _CH
    npair = nch // 2
    mesh = plsc.VectorSubcoreMesh(core_axis_name="c", subcore_axis_name="s")

    @functools.partial(
        pl.kernel,
        out_type=jax.ShapeDtypeStruct((E, C), jnp.float32),
        mesh=mesh,
        scratch_types=[
            pltpu.VMEM((per_w,), jnp.int32),
            pltpu.VMEM((_CH, C), jnp.float32),
            pltpu.VMEM((_CH, C), jnp.float32),
            pltpu.SemaphoreType.DMA,
            pltpu.SemaphoreType.DMA,
        ],
    )
    def k(table_hbm, idx_hbm, out_hbm, idx_v, buf0, buf1, sem0, sem1):
        wid = lax.axis_index("s") * _NC + lax.axis_index("c")
        base = wid * per_w
        pltpu.sync_copy(idx_hbm.at[pl.ds(base, per_w)], idx_v)

        def step(i, carry):
            r0 = i * (2 * _CH)
            cp0 = pltpu.async_copy(table_hbm.at[idx_v.at[pl.ds(r0, _CH)]], buf0, sem0)
            cp1 = pltpu.async_copy(
                table_hbm.at[idx_v.at[pl.ds(r0 + _CH, _CH)]], buf1, sem1)
            cp0.wait()
            pltpu.sync_copy(buf0, out_hbm.at[pl.ds(base + r0, _CH)])
            cp1.wait()
            pltpu.sync_copy(buf1, out_hbm.at[pl.ds(base + r0 + _CH, _CH)])
            return carry

        lax.fori_loop(0, npair, step, 0)
        if nch % 2:
            r0 = (nch - 1) * _CH
            pltpu.async_copy(table_hbm.at[idx_v.at[pl.ds(r0, _CH)]], buf0, sem0).wait()
            pltpu.sync_copy(buf0, out_hbm.at[pl.ds(base + r0, _CH)])

    return k(table, idx)


def _sc_gather2(table, idx_a, idx_b):
    C = table.shape[1]
    E = idx_a.shape[0]
    per_w = E // _NW
    nch = per_w // _CH
    mesh = plsc.VectorSubcoreMesh(core_axis_name="c", subcore_axis_name="s")

    @functools.partial(
        pl.kernel,
        out_type=jax.ShapeDtypeStruct((2, E, C), jnp.float32),
        mesh=mesh,
        scratch_types=[
            pltpu.VMEM((per_w,), jnp.int32),
            pltpu.VMEM((per_w,), jnp.int32),
            pltpu.VMEM((_CH, C), jnp.float32),
            pltpu.VMEM((_CH, C), jnp.float32),
            pltpu.SemaphoreType.DMA,
            pltpu.SemaphoreType.DMA,
        ],
    )
    def k(table_hbm, ia_hbm, ib_hbm, out_hbm, ia_v, ib_v, buf0, buf1, sem0, sem1):
        wid = lax.axis_index("s") * _NC + lax.axis_index("c")
        base = wid * per_w
        pltpu.sync_copy(ia_hbm.at[pl.ds(base, per_w)], ia_v)
        pltpu.sync_copy(ib_hbm.at[pl.ds(base, per_w)], ib_v)

        def step(i, carry):
            r0 = i * _CH
            cp0 = pltpu.async_copy(table_hbm.at[ia_v.at[pl.ds(r0, _CH)]], buf0, sem0)
            cp1 = pltpu.async_copy(table_hbm.at[ib_v.at[pl.ds(r0, _CH)]], buf1, sem1)
            cp0.wait()
            pltpu.sync_copy(buf0, out_hbm.at[0, pl.ds(base + r0, _CH)])
            cp1.wait()
            pltpu.sync_copy(buf1, out_hbm.at[1, pl.ds(base + r0, _CH)])
            return carry

        lax.fori_loop(0, nch, step, 0)

    return k(table, idx_a, idx_b)


def _sc_scatter_dual(data, idxs, zeros_np, c0_pair, C, npad):
    E = data.shape[0]
    per_w = E // _NS
    nch = per_w // _CH
    NPH = 2
    nph_ch = nch // NPH
    rows_t = npad // _NS
    c0a, c0b = c0_pair
    idx_per_core = idxs.ndim == 5
    mesh = plsc.VectorSubcoreMesh(core_axis_name="c", subcore_axis_name="s")

    @functools.partial(
        pl.kernel,
        out_type=jax.ShapeDtypeStruct((2, npad, C), jnp.float32),
        mesh=mesh,
        scratch_types=[
            pltpu.VMEM((nph_ch, _CH), jnp.int32),
            pltpu.VMEM((_CH, C), jnp.float32),
            pltpu.VMEM((_CH, C), jnp.float32),
            pltpu.VMEM_SHARED((npad, C), jnp.float32),
            pltpu.SemaphoreType.DMA,
            pltpu.SemaphoreType.DMA,
        ],
    )
    def k(data_hbm, idx_hbm, z_hbm, out_hbm, idx_v, dbuf0, dbuf1, accum,
          sem0, sem1):
        c_ax = lax.axis_index("c")
        s_ax = lax.axis_index("s")
        base = s_ax * per_w
        row0 = s_ax * rows_t
        if c0a == c0b:
            c0 = c0a
        else:
            c0 = c0a + c_ax * (c0b - c0a)
        pltpu.sync_copy(z_hbm.at[pl.ds(row0, rows_t)],
                        accum.at[pl.ds(row0, rows_t)])
        plsc.subcore_barrier()
        for ph in range(NPH):
            if idx_per_core:
                pltpu.sync_copy(idx_hbm.at[c_ax, s_ax, ph], idx_v)
            else:
                pltpu.sync_copy(idx_hbm.at[s_ax, ph], idx_v)
            pbase = base + ph * nph_ch * _CH

            def step(i, carry):
                j0 = i * 2
                cp0 = pltpu.async_copy(
                    data_hbm.at[pl.ds(pbase + j0 * _CH, _CH), pl.ds(c0, C)],
                    dbuf0, sem0)
                cp1 = pltpu.async_copy(
                    data_hbm.at[pl.ds(pbase + (j0 + 1) * _CH, _CH),
                                pl.ds(c0, C)],
                    dbuf1, sem1)
                cp0.wait()
                pltpu.sync_copy(dbuf0, accum.at[idx_v.at[j0]], add=True)
                cp1.wait()
                pltpu.sync_copy(dbuf1, accum.at[idx_v.at[j0 + 1]], add=True)
                return carry

            lax.fori_loop(0, nph_ch // 2, step, 0)
            if nph_ch % 2:
                j = nph_ch - 1
                pltpu.async_copy(
                    data_hbm.at[pl.ds(pbase + j * _CH, _CH), pl.ds(c0, C)],
                    dbuf0, sem0).wait()
                pltpu.sync_copy(dbuf0, accum.at[idx_v.at[j]], add=True)
        plsc.subcore_barrier()
        pltpu.sync_copy(accum.at[pl.ds(row0, rows_t)],
                        out_hbm.at[c_ax, pl.ds(row0, rows_t)])

    return k(data, idxs, zeros_np)


def _sc_scatter_add(data, idx3, zeros_np, c0, C, npad):
    E = data.shape[0]
    per_w = E // _NW
    nch = per_w // _CH
    rows_t = npad // _NS
    mesh = plsc.VectorSubcoreMesh(core_axis_name="c", subcore_axis_name="s")

    @functools.partial(
        pl.kernel,
        out_type=jax.ShapeDtypeStruct((2, npad, C), jnp.float32),
        mesh=mesh,
        scratch_types=[
            pltpu.VMEM((nch, _CH), jnp.int32),
            pltpu.VMEM((_CH, C), jnp.float32),
            pltpu.VMEM((_CH, C), jnp.float32),
            pltpu.VMEM_SHARED((npad, C), jnp.float32),
            pltpu.SemaphoreType.DMA,
            pltpu.SemaphoreType.DMA,
        ],
    )
    def k(data_hbm, idx_hbm, z_hbm, out_hbm, idx_v, dbuf0, dbuf1, accum,
          sem0, sem1):
        c_ax = lax.axis_index("c")
        s_ax = lax.axis_index("s")
        wid = s_ax * _NC + c_ax
        base = wid * per_w
        row0 = s_ax * rows_t
        pltpu.sync_copy(z_hbm.at[pl.ds(row0, rows_t)],
                        accum.at[pl.ds(row0, rows_t)])
        pltpu.sync_copy(idx_hbm.at[wid], idx_v)
        plsc.subcore_barrier()

        def step(i, carry):
            j0 = i * 2
            cp0 = pltpu.async_copy(
                data_hbm.at[pl.ds(base + j0 * _CH, _CH), pl.ds(c0, C)],
                dbuf0, sem0)
            cp1 = pltpu.async_copy(
                data_hbm.at[pl.ds(base + (j0 + 1) * _CH, _CH), pl.ds(c0, C)],
                dbuf1, sem1)
            cp0.wait()
            pltpu.sync_copy(dbuf0, accum.at[idx_v.at[j0]], add=True)
            cp1.wait()
            pltpu.sync_copy(dbuf1, accum.at[idx_v.at[j0 + 1]], add=True)
            return carry

        lax.fori_loop(0, nch // 2, step, 0)
        if nch % 2:
            j = nch - 1
            pltpu.async_copy(
                data_hbm.at[pl.ds(base + j * _CH, _CH), pl.ds(c0, C)],
                dbuf0, sem0).wait()
            pltpu.sync_copy(dbuf0, accum.at[idx_v.at[j]], add=True)
        plsc.subcore_barrier()
        pltpu.sync_copy(accum.at[pl.ds(row0, rows_t)],
                        out_hbm.at[c_ax, pl.ds(row0, rows_t)])

    return k(data, idx3, zeros_np)


def kernel(x, pos, edge_index, period_vec, batch, elem_embed, W_embed, b_embed,
           rbf_centers, W_msg, W_filter, W_self, W_attr, w_sh, b_conv,
           W_post0, b_post0, W_post1, b_post1, scale, shift):
    N = pos.shape[0]
    E = edge_index.shape[1]
    DA = elem_embed.shape[1]
    D = W_embed.shape[1]
    K = rbf_centers.shape[0]
    DH = W_post0.shape[1]
    T = W_post1.shape[1]

    NB = 1024
    NPAD = ((N + NB - 1) // NB) * NB
    EB = _blk(E)

    src = edge_index[0].astype(jnp.int32)
    dst = edge_index[1].astype(jnp.int32)
    xi_p = jnp.pad(x.reshape(-1, 1).astype(jnp.int32), ((0, NPAD - N), (0, 0)))
    batch_p = jnp.pad(batch.astype(jnp.int32).reshape(-1, 1),
                      ((0, NPAD - N), (0, 0)), constant_values=G_SEGMENTS)
    pos128 = jnp.pad(pos, ((0, NPAD - N), (0, 125)))
    per_w = E // _NW
    nch = per_w // _CH
    dst3 = dst.reshape(_NW, nch, _CH)
    nch2 = (E // _NS) // _CH
    src16 = src.reshape(_NS, 2, nch2 // 2, _CH)
    dst16 = dst.reshape(_NS, 2, nch2 // 2, _CH)
    ds4 = jnp.stack([dst16, src16])
    z128 = jnp.zeros((NPAD, D), jnp.float32)
    z8 = jnp.zeros((NPAD, 8), jnp.float32)

    NE_PAD = 128
    elem_pad = jnp.pad(elem_embed, ((0, NE_PAD - elem_embed.shape[0]), (0, 0)))
    centers = rbf_centers.reshape(1, K)
    b_embed2 = b_embed.reshape(1, D)
    b0_2 = b_post0.reshape(1, DH)
    b1_2 = b_post1.reshape(1, T)
    z1bar = scale[0][:, None] * W_post1.T
    W0T = W_post0.T
    WmT = [W_msg[l].T for l in range(3)]
    WfT = [W_filter[l].T for l in range(3)]
    WsT = [W_self[l].T for l in range(3)]

    x_attr, h0 = _t1_embed(xi_p, elem_pad, W_embed, b_embed2, NB)
    pg = _sc_gather2(pos128, dst, src)
    geom = _t_geom(pg, period_vec, w_sh, EB)

    hs_l = []
    pre_l = []
    h = h0
    for l in range(3):
        hs = _sc_gather(h, src)
        hs_l.append(hs)
        m = _t2_edge_fwd(hs, geom, W_msg[l], W_filter[l], centers, l, EB)
        agg2 = _sc_scatter_add(m, dst3, z128, 0, D, NPAD)
        pre, h = _t3_node(agg2, h, x_attr, W_self[l], W_attr[l],
                          b_conv[l].reshape(1, D), NB)
        pre_l.append(pre)

    z0, energies = _t4_head(h, batch_p, W_post0, b0_2, W_post1, b1_2,
                            scale, shift, NB)

    hbar = _t5_head_bwd(z0, z1bar, W0T, NB)
    gacc = jnp.zeros((E, 8), jnp.float32)
    hbar_halves = None
    for l in range(2, -1, -1):
        prebar, hbar_self = _t6_node_bwd(hbar, hbar_halves, pre_l[l], WsT[l],
                                         NB, need_self=(l > 0))
        mbar = _sc_gather(prebar, dst)
        outsc, gacc = _t7_edge_bwd(mbar, hs_l[l], geom, gacc, W_msg[l],
                                   W_filter[l], WmT[l], WfT[l],
                                   w_sh[l].reshape(1, 4), centers, l, EB,
                                   need_scatter=(l > 0))
        if l > 0:
            hbar = hbar_self
            hbar_halves = _sc_scatter_dual(outsc, src16, z128, (0, D), D, NPAD)

    evb = _t8_geom_bwd(geom, gacc, EB)
    pb = _sc_scatter_dual(evb, ds4, z128, (0, 0), D, NPAD)
    fb = _t9_forces(pb, NB)

    forces = jnp.stack([fb[:N, 0:3], fb[:N, 4:7]], axis=1)
    return (energies, forces)


# speedup vs baseline: 2.8177x; 1.0139x over previous
import functools

import jax
import jax.numpy as jnp
from jax import lax
from jax.experimental import pallas as pl
from jax.experimental.pallas import tpu as pltpu
from jax.experimental.pallas import tpu_sc as plsc

_NC, _NS = 2, 16
_NW = _NC * _NS
_CH = 80

GAMMA = 10.0
EPS = 1e-12
G_SEGMENTS = 64


def _silu(x):
    return x * jax.nn.sigmoid(x)


def _dsilu(x):
    s = jax.nn.sigmoid(x)
    return s * (1.0 + x * (1.0 - s))


def _blk(E):
    for eb in (2000, 1600, 1000, 800, 500, 400, 200, 100, 8):
        if E % eb == 0:
            return eb
    return E


def _t1_embed(xi, elem_pad, W_embed, b_embed, NB):
    NPAD = xi.shape[0]
    DA = elem_pad.shape[1]
    D = W_embed.shape[1]
    NE = elem_pad.shape[0]

    def body(xi_ref, elem_ref, we_ref, be_ref, xa_ref, h0_ref):
        ids = xi_ref[...]
        cols = lax.broadcasted_iota(jnp.int32, (NB, NE), 1)
        oh = (cols == ids).astype(jnp.float32)
        xa = jnp.dot(oh, elem_ref[...], preferred_element_type=jnp.float32)
        xa_ref[...] = xa
        h0_ref[...] = jnp.dot(xa, we_ref[...], preferred_element_type=jnp.float32) + be_ref[...]

    return pl.pallas_call(
        body,
        grid=(NPAD // NB,),
        in_specs=[
            pl.BlockSpec((NB, 1), lambda i: (i, 0)),
            pl.BlockSpec((NE, DA), lambda i: (0, 0)),
            pl.BlockSpec((DA, D), lambda i: (0, 0)),
            pl.BlockSpec((1, D), lambda i: (0, 0)),
        ],
        out_specs=[
            pl.BlockSpec((NB, DA), lambda i: (i, 0)),
            pl.BlockSpec((NB, D), lambda i: (i, 0)),
        ],
        out_shape=[
            jax.ShapeDtypeStruct((NPAD, DA), jnp.float32),
            jax.ShapeDtypeStruct((NPAD, D), jnp.float32),
        ],
    )(xi, elem_pad, W_embed, b_embed)


def _t_geom(pg, pv, w_sh, EB):
    E = pg.shape[1]

    def body(pg_ref, pv_ref, wsh_ref, out_ref):
        ev = pg_ref[0][:, 0:3] - pg_ref[1][:, 0:3] + pv_ref[...]
        r = jnp.sqrt(jnp.sum(ev * ev, axis=1, keepdims=True) + EPS)
        u = ev / r
        wsh = wsh_ref[...]
        s_all = jnp.dot(u, wsh[:, 1:4].T, preferred_element_type=jnp.float32) + wsh[:, 0][None, :]
        z = jnp.zeros((ev.shape[0], 1), jnp.float32)
        out_ref[...] = jnp.concatenate([s_all, r, u, z], axis=1)

    return pl.pallas_call(
        body,
        grid=(E // EB,),
        in_specs=[
            pl.BlockSpec((2, EB, 128), lambda i: (0, i, 0)),
            pl.BlockSpec((EB, 3), lambda i: (i, 0)),
            pl.BlockSpec((3, 4), lambda i: (0, 0)),
        ],
        out_specs=pl.BlockSpec((EB, 8), lambda i: (i, 0)),
        out_shape=jax.ShapeDtypeStruct((E, 8), jnp.float32),
    )(pg, pv, w_sh)


def _t2_edge_fwd(hs, geom, Wm, Wf, centers, l, EB):
    E, D = hs.shape
    K = Wf.shape[0]

    def body(hs_ref, g_ref, wm_ref, wf_ref, c_ref, m_ref):
        geomb = g_ref[...]
        r = geomb[:, 3:4]
        s = geomb[:, l:l + 1]
        rbf = jnp.exp(-GAMMA * (r - c_ref[...]) ** 2)
        filt = jnp.dot(rbf, wf_ref[...], preferred_element_type=jnp.float32)
        gg = jnp.dot(hs_ref[...], wm_ref[...], preferred_element_type=jnp.float32)
        m_ref[...] = gg * filt * s

    return pl.pallas_call(
        body,
        grid=(E // EB,),
        in_specs=[
            pl.BlockSpec((EB, D), lambda i: (i, 0)),
            pl.BlockSpec((EB, 8), lambda i: (i, 0)),
            pl.BlockSpec((D, D), lambda i: (0, 0)),
            pl.BlockSpec((K, D), lambda i: (0, 0)),
            pl.BlockSpec((1, K), lambda i: (0, 0)),
        ],
        out_specs=pl.BlockSpec((EB, D), lambda i: (i, 0)),
        out_shape=jax.ShapeDtypeStruct((E, D), jnp.float32),
    )(hs, geom, Wm, Wf, centers)


def _t3_node(agg2, h, x_attr, Ws, Wa, bc, NB):
    NPAD, D = h.shape
    DA = x_attr.shape[1]

    def body(a_ref, h_ref, xa_ref, ws_ref, wa_ref, bc_ref, pre_ref, hn_ref):
        pre = (a_ref[0] + a_ref[1]
               + jnp.dot(h_ref[...], ws_ref[...], preferred_element_type=jnp.float32)
               + jnp.dot(xa_ref[...], wa_ref[...], preferred_element_type=jnp.float32)
               + bc_ref[...])
        pre_ref[...] = pre
        hn_ref[...] = _silu(pre)

    return pl.pallas_call(
        body,
        grid=(NPAD // NB,),
        in_specs=[
            pl.BlockSpec((2, NB, D), lambda i: (0, i, 0)),
            pl.BlockSpec((NB, D), lambda i: (i, 0)),
            pl.BlockSpec((NB, DA), lambda i: (i, 0)),
            pl.BlockSpec((D, D), lambda i: (0, 0)),
            pl.BlockSpec((DA, D), lambda i: (0, 0)),
            pl.BlockSpec((1, D), lambda i: (0, 0)),
        ],
        out_specs=[
            pl.BlockSpec((NB, D), lambda i: (i, 0)),
            pl.BlockSpec((NB, D), lambda i: (i, 0)),
        ],
        out_shape=[
            jax.ShapeDtypeStruct((NPAD, D), jnp.float32),
            jax.ShapeDtypeStruct((NPAD, D), jnp.float32),
        ],
    )(agg2, h, x_attr, Ws, Wa, bc)


def _t4_head(h, batch_p, W0, b0, W1, b1, scale, shift, NB):
    NPAD, D = h.shape
    DH = W0.shape[1]
    T = W1.shape[1]
    nblocks = NPAD // NB

    def body(h_ref, b_ref, w0_ref, b0_ref, w1_ref, b1_ref, sc_ref, sh_ref,
             z0_ref, en_ref):
        i = pl.program_id(0)
        z0 = jnp.dot(h_ref[...], w0_ref[...], preferred_element_type=jnp.float32) + b0_ref[...]
        z0_ref[...] = z0
        o = jnp.dot(_silu(z0), w1_ref[...], preferred_element_type=jnp.float32) + b1_ref[...]
        seg = b_ref[...]
        rows = lax.broadcasted_iota(jnp.int32, (G_SEGMENTS, NB), 0)
        oh = (rows == seg[:, 0][None, :]).astype(jnp.float32)
        part = jnp.dot(oh, o, preferred_element_type=jnp.float32)

        @pl.when(i == 0)
        def _():
            en_ref[...] = jnp.zeros_like(en_ref)

        en_ref[...] += part

        @pl.when(i == nblocks - 1)
        def _():
            en_ref[...] = en_ref[...] * sc_ref[...] + sh_ref[...]

    return pl.pallas_call(
        body,
        grid=(nblocks,),
        in_specs=[
            pl.BlockSpec((NB, D), lambda i: (i, 0)),
            pl.BlockSpec((NB, 1), lambda i: (i, 0)),
            pl.BlockSpec((D, DH), lambda i: (0, 0)),
            pl.BlockSpec((1, DH), lambda i: (0, 0)),
            pl.BlockSpec((DH, T), lambda i: (0, 0)),
            pl.BlockSpec((1, T), lambda i: (0, 0)),
            pl.BlockSpec((1, T), lambda i: (0, 0)),
            pl.BlockSpec((1, T), lambda i: (0, 0)),
        ],
        out_specs=[
            pl.BlockSpec((NB, DH), lambda i: (i, 0)),
            pl.BlockSpec((G_SEGMENTS, T), lambda i: (0, 0)),
        ],
        out_shape=[
            jax.ShapeDtypeStruct((NPAD, DH), jnp.float32),
            jax.ShapeDtypeStruct((G_SEGMENTS, T), jnp.float32),
        ],
    )(h, batch_p, W0, b0, W1, b1, scale, shift)


def _t5_head_bwd(z0, z1bar, W0T, NB):
    NPAD, DH = z0.shape
    D = W0T.shape[1]

    def body(z0_ref, zb_ref, w0t_ref, hb_ref):
        d = _dsilu(z0_ref[...])
        zb = zb_ref[...]
        h0 = jnp.dot(d * zb[0][None, :], w0t_ref[...], preferred_element_type=jnp.float32)
        h1 = jnp.dot(d * zb[1][None, :], w0t_ref[...], preferred_element_type=jnp.float32)
        hb_ref[...] = jnp.concatenate([h0, h1], axis=1)

    return pl.pallas_call(
        body,
        grid=(NPAD // NB,),
        in_specs=[
            pl.BlockSpec((NB, DH), lambda i: (i, 0)),
            pl.BlockSpec((2, DH), lambda i: (0, 0)),
            pl.BlockSpec((DH, D), lambda i: (0, 0)),
        ],
        out_specs=pl.BlockSpec((NB, 2 * D), lambda i: (i, 0)),
        out_shape=jax.ShapeDtypeStruct((NPAD, 2 * D), jnp.float32),
    )(z0, z1bar, W0T)


def _t6_node_bwd(hbar_full, hbar_halves, pre, WsT, NB, need_self):
    NPAD, D = pre.shape
    have_halves = hbar_halves is not None

    def body(*refs):
        i = 0
        hf_ref = refs[i]; i += 1
        if have_halves:
            ha_ref = refs[i]; i += 1
        pre_ref = refs[i]; wst_ref = refs[i + 1]; i += 2
        pb_ref = refs[i]; i += 1
        acc = hf_ref[...]
        if have_halves:
            acc = acc + jnp.concatenate([ha_ref[0], ha_ref[1]], axis=1)
        d = _dsilu(pre_ref[...])
        prebar = acc * jnp.concatenate([d, d], axis=1)
        pb_ref[...] = prebar
        if need_self:
            hs_ref = refs[i]
            wst = wst_ref[...]
            h0 = jnp.dot(prebar[:, :D], wst, preferred_element_type=jnp.float32)
            h1 = jnp.dot(prebar[:, D:], wst, preferred_element_type=jnp.float32)
            hs_ref[...] = jnp.concatenate([h0, h1], axis=1)

    in_specs = [pl.BlockSpec((NB, 2 * D), lambda i: (i, 0))]
    ins = [hbar_full]
    if have_halves:
        in_specs += [pl.BlockSpec((2, NB, D), lambda i: (0, i, 0))]
        ins += [hbar_halves]
    in_specs += [
        pl.BlockSpec((NB, D), lambda i: (i, 0)),
        pl.BlockSpec((D, D), lambda i: (0, 0)),
    ]
    ins += [pre, WsT]

    out_specs = [pl.BlockSpec((NB, 2 * D), lambda i: (i, 0))]
    out_shape = [jax.ShapeDtypeStruct((NPAD, 2 * D), jnp.float32)]
    if need_self:
        out_specs.append(pl.BlockSpec((NB, 2 * D), lambda i: (i, 0)))
        out_shape.append(jax.ShapeDtypeStruct((NPAD, 2 * D), jnp.float32))

    res = pl.pallas_call(
        body,
        grid=(NPAD // NB,),
        in_specs=in_specs,
        out_specs=out_specs,
        out_shape=out_shape,
    )(*ins)
    return res if need_self else (res[0], None)


def _t7_edge_bwd(mbar, hs, geom, gacc, Wm, Wf, WmT, WfT, wsh_l, centers, l, EB,
                 need_scatter):
    E, D = hs.shape
    K = Wf.shape[0]

    def body(mb_ref, hs_ref, g_ref, ga_ref, wm_ref, wf_ref, wmt_ref, wft_ref,
             wshl_ref, c_ref, *outs):
        geomb = g_ref[...]
        r = geomb[:, 3:4]
        s = geomb[:, l:l + 1]
        cen = c_ref[...]
        rbf = jnp.exp(-GAMMA * (r - cen) ** 2)
        drbf = (-2.0 * GAMMA) * (r - cen) * rbf
        filt = jnp.dot(rbf, wf_ref[...], preferred_element_type=jnp.float32)
        g = jnp.dot(hs_ref[...], wm_ref[...], preferred_element_type=jnp.float32)
        mb = mb_ref[...]
        wshl = wshl_ref[...]
        fs = filt * s
        acc_parts = []
        sc_parts = []
        for i in range(2):
            mbi = mb[:, i * D:(i + 1) * D]
            if need_scatter:
                gbar = mbi * fs
                sc_parts.append(jnp.dot(gbar, wmt_ref[...], preferred_element_type=jnp.float32))
            mg = mbi * g
            fbar = jnp.dot(mg * s, wft_ref[...], preferred_element_type=jnp.float32)
            rbar = jnp.sum(fbar * drbf, axis=1, keepdims=True)
            sb = jnp.sum(mg * filt, axis=1, keepdims=True)
            acc_parts.append(jnp.concatenate(
                [rbar, sb * wshl[0, 1], sb * wshl[0, 2], sb * wshl[0, 3]], axis=1))
        ga_out = outs[-1]
        ga_total = ga_ref[...] + jnp.concatenate(acc_parts, axis=1)
        if need_scatter:
            ga_out[...] = ga_total
            outs[0][...] = jnp.concatenate(sc_parts, axis=1)
        else:
            u = geomb[:, 4:7]
            z1 = jnp.zeros((geomb.shape[0], 1), jnp.float32)
            ev_parts = []
            for i in range(2):
                rb = ga_total[:, 4 * i:4 * i + 1]
                ub = ga_total[:, 4 * i + 1:4 * i + 4]
                uu = jnp.sum(ub * u, axis=1, keepdims=True)
                evb = (ub - u * uu) / r + u * rb
                ev_parts.extend([evb, z1])
            ztail = jnp.zeros((geomb.shape[0], 120), jnp.float32)
            ga_out[...] = jnp.concatenate(ev_parts + [ztail], axis=1)

    out_specs = []
    out_shape = []
    if need_scatter:
        out_specs.append(pl.BlockSpec((EB, 2 * D), lambda i: (i, 0)))
        out_shape.append(jax.ShapeDtypeStruct((E, 2 * D), jnp.float32))
        out_specs.append(pl.BlockSpec((EB, 8), lambda i: (i, 0)))
        out_shape.append(jax.ShapeDtypeStruct((E, 8), jnp.float32))
    else:
        out_specs.append(pl.BlockSpec((EB, 128), lambda i: (i, 0)))
        out_shape.append(jax.ShapeDtypeStruct((E, 128), jnp.float32))

    res = pl.pallas_call(
        body,
        grid=(E // EB,),
        in_specs=[
            pl.BlockSpec((EB, 2 * D), lambda i: (i, 0)),
            pl.BlockSpec((EB, D), lambda i: (i, 0)),
            pl.BlockSpec((EB, 8), lambda i: (i, 0)),
            pl.BlockSpec((EB, 8), lambda i: (i, 0)),
            pl.BlockSpec((D, D), lambda i: (0, 0)),
            pl.BlockSpec((K, D), lambda i: (0, 0)),
            pl.BlockSpec((D, D), lambda i: (0, 0)),
            pl.BlockSpec((D, K), lambda i: (0, 0)),
            pl.BlockSpec((1, 4), lambda i: (0, 0)),
            pl.BlockSpec((1, K), lambda i: (0, 0)),
        ],
        out_specs=out_specs,
        out_shape=out_shape,
    )(mbar, hs, geom, gacc, Wm, Wf, WmT, WfT, wsh_l, centers)
    if need_scatter:
        return res[0], res[1]
    return None, res[0]


def _t8_geom_bwd(geom, gacc, EB):
    E = geom.shape[0]

    def body(g_ref, ga_ref, out_ref):
        geomb = g_ref[...]
        ga = ga_ref[...]
        r = geomb[:, 3:4]
        u = geomb[:, 4:7]
        z = jnp.zeros((geomb.shape[0], 1), jnp.float32)
        parts = []
        for i in range(2):
            rb = ga[:, 4 * i:4 * i + 1]
            ub = ga[:, 4 * i + 1:4 * i + 4]
            uu = jnp.sum(ub * u, axis=1, keepdims=True)
            evb = (ub - u * uu) / r + u * rb
            parts.extend([evb, z])
        ztail = jnp.zeros((geomb.shape[0], 120), jnp.float32)
        out_ref[...] = jnp.concatenate(parts + [ztail], axis=1)

    return pl.pallas_call(
        body,
        grid=(E // EB,),
        in_specs=[
            pl.BlockSpec((EB, 8), lambda i: (i, 0)),
            pl.BlockSpec((EB, 8), lambda i: (i, 0)),
        ],
        out_specs=pl.BlockSpec((EB, 128), lambda i: (i, 0)),
        out_shape=jax.ShapeDtypeStruct((E, 128), jnp.float32),
    )(geom, gacc)


def _t9_forces(pb, NB):
    NPAD = pb.shape[1]

    def body(pb_ref, out_ref):
        f = pb_ref[1] - pb_ref[0]
        out_ref[...] = f[:, 0:8]

    return pl.pallas_call(
        body,
        grid=(NPAD // NB,),
        in_specs=[pl.BlockSpec((2, NB, 128), lambda i: (0, i, 0))],
        out_specs=pl.BlockSpec((NB, 8), lambda i: (i, 0)),
        out_shape=jax.ShapeDtypeStruct((NPAD, 8), jnp.float32),
    )(pb)


def _sc_gather(table, idx):
    C = table.shape[1]
    E = idx.shape[0]
    per_w = E // _NW
    nch = per_w // _CH
    npair = nch // 2
    mesh = plsc.VectorSubcoreMesh(core_axis_name="c", subcore_axis_name="s")

    @functools.partial(
        pl.kernel,
        out_type=jax.ShapeDtypeStruct((E, C), jnp.float32),
        mesh=mesh,
        scratch_types=[
            pltpu.VMEM((per_w,), jnp.int32),
            pltpu.VMEM((_CH, C), jnp.float32),
            pltpu.VMEM((_CH, C), jnp.float32),
            pltpu.SemaphoreType.DMA,
            pltpu.SemaphoreType.DMA,
        ],
    )
    def k(table_hbm, idx_hbm, out_hbm, idx_v, buf0, buf1, sem0, sem1):
        wid = lax.axis_index("s") * _NC + lax.axis_index("c")
        base = wid * per_w
        pltpu.sync_copy(idx_hbm.at[pl.ds(base, per_w)], idx_v)

        def step(i, carry):
            r0 = i * (2 * _CH)
            cp0 = pltpu.async_copy(table_hbm.at[idx_v.at[pl.ds(r0, _CH)]], buf0, sem0)
            cp1 = pltpu.async_copy(
                table_hbm.at[idx_v.at[pl.ds(r0 + _CH, _CH)]], buf1, sem1)
            cp0.wait()
            pltpu.sync_copy(buf0, out_hbm.at[pl.ds(base + r0, _CH)])
            cp1.wait()
            pltpu.sync_copy(buf1, out_hbm.at[pl.ds(base + r0 + _CH, _CH)])
            return carry

        lax.fori_loop(0, npair, step, 0)
        if nch % 2:
            r0 = (nch - 1) * _CH
            pltpu.async_copy(table_hbm.at[idx_v.at[pl.ds(r0, _CH)]], buf0, sem0).wait()
            pltpu.sync_copy(buf0, out_hbm.at[pl.ds(base + r0, _CH)])

    return k(table, idx)


def _sc_gather2(table, idx_a, idx_b):
    C = table.shape[1]
    E = idx_a.shape[0]
    per_w = E // _NW
    nch = per_w // _CH
    mesh = plsc.VectorSubcoreMesh(core_axis_name="c", subcore_axis_name="s")

    @functools.partial(
        pl.kernel,
        out_type=jax.ShapeDtypeStruct((2, E, C), jnp.float32),
        mesh=mesh,
        scratch_types=[
            pltpu.VMEM((per_w,), jnp.int32),
            pltpu.VMEM((per_w,), jnp.int32),
            pltpu.VMEM((_CH, C), jnp.float32),
            pltpu.VMEM((_CH, C), jnp.float32),
            pltpu.SemaphoreType.DMA,
            pltpu.SemaphoreType.DMA,
        ],
    )
    def k(table_hbm, ia_hbm, ib_hbm, out_hbm, ia_v, ib_v, buf0, buf1, sem0, sem1):
        wid = lax.axis_index("s") * _NC + lax.axis_index("c")
        base = wid * per_w
        pltpu.sync_copy(ia_hbm.at[pl.ds(base, per_w)], ia_v)
        pltpu.sync_copy(ib_hbm.at[pl.ds(base, per_w)], ib_v)

        def step(i, carry):
            r0 = i * _CH
            cp0 = pltpu.async_copy(table_hbm.at[ia_v.at[pl.ds(r0, _CH)]], buf0, sem0)
            cp1 = pltpu.async_copy(table_hbm.at[ib_v.at[pl.ds(r0, _CH)]], buf1, sem1)
            cp0.wait()
            pltpu.sync_copy(buf0, out_hbm.at[0, pl.ds(base + r0, _CH)])
            cp1.wait()
            pltpu.sync_copy(buf1, out_hbm.at[1, pl.ds(base + r0, _CH)])
            return carry

        lax.fori_loop(0, nch, step, 0)

    return k(table, idx_a, idx_b)


def _sc_scatter_dual(data, idxs, zeros_np, c0_pair, C, npad):
    E = data.shape[0]
    per_w = E // _NS
    nch = per_w // _CH
    NPH = 2
    nph_ch = nch // NPH
    rows_t = npad // _NS
    c0a, c0b = c0_pair
    idx_per_core = idxs.ndim == 5
    mesh = plsc.VectorSubcoreMesh(core_axis_name="c", subcore_axis_name="s")

    @functools.partial(
        pl.kernel,
        out_type=jax.ShapeDtypeStruct((2, npad, C), jnp.float32),
        mesh=mesh,
        scratch_types=[
            pltpu.VMEM((nph_ch, _CH), jnp.int32),
            pltpu.VMEM((_CH, C), jnp.float32),
            pltpu.VMEM((_CH, C), jnp.float32),
            pltpu.VMEM_SHARED((npad, C), jnp.float32),
            pltpu.SemaphoreType.DMA,
            pltpu.SemaphoreType.DMA,
        ],
    )
    def k(data_hbm, idx_hbm, z_hbm, out_hbm, idx_v, dbuf0, dbuf1, accum,
          sem0, sem1):
        c_ax = lax.axis_index("c")
        s_ax = lax.axis_index("s")
        base = s_ax * per_w
        row0 = s_ax * rows_t
        if c0a == c0b:
            c0 = c0a
        else:
            c0 = c0a + c_ax * (c0b - c0a)
        pltpu.sync_copy(z_hbm.at[pl.ds(row0, rows_t)],
                        accum.at[pl.ds(row0, rows_t)])
        plsc.subcore_barrier()
        for ph in range(NPH):
            if idx_per_core:
                pltpu.sync_copy(idx_hbm.at[c_ax, s_ax, ph], idx_v)
            else:
                pltpu.sync_copy(idx_hbm.at[s_ax, ph], idx_v)
            pbase = base + ph * nph_ch * _CH

            def step(i, carry):
                j0 = i * 2
                cp0 = pltpu.async_copy(
                    data_hbm.at[pl.ds(pbase + j0 * _CH, _CH), pl.ds(c0, C)],
                    dbuf0, sem0)
                cp1 = pltpu.async_copy(
                    data_hbm.at[pl.ds(pbase + (j0 + 1) * _CH, _CH),
                                pl.ds(c0, C)],
                    dbuf1, sem1)
                cp0.wait()
                pltpu.sync_copy(dbuf0, accum.at[idx_v.at[j0]], add=True)
                cp1.wait()
                pltpu.sync_copy(dbuf1, accum.at[idx_v.at[j0 + 1]], add=True)
                return carry

            lax.fori_loop(0, nph_ch // 2, step, 0)
            if nph_ch % 2:
                j = nph_ch - 1
                pltpu.async_copy(
                    data_hbm.at[pl.ds(pbase + j * _CH, _CH), pl.ds(c0, C)],
                    dbuf0, sem0).wait()
                pltpu.sync_copy(dbuf0, accum.at[idx_v.at[j]], add=True)
        plsc.subcore_barrier()
        pltpu.sync_copy(accum.at[pl.ds(row0, rows_t)],
                        out_hbm.at[c_ax, pl.ds(row0, rows_t)])

    return k(data, idxs, zeros_np)


def _sc_scatter_add(data, idx3, zeros_np, c0, C, npad):
    E = data.shape[0]
    per_w = E // _NW
    nch = per_w // _CH
    rows_t = npad // _NS
    mesh = plsc.VectorSubcoreMesh(core_axis_name="c", subcore_axis_name="s")

    @functools.partial(
        pl.kernel,
        out_type=jax.ShapeDtypeStruct((2, npad, C), jnp.float32),
        mesh=mesh,
        scratch_types=[
            pltpu.VMEM((nch, _CH), jnp.int32),
            pltpu.VMEM((_CH, C), jnp.float32),
            pltpu.VMEM((_CH, C), jnp.float32),
            pltpu.VMEM_SHARED((npad, C), jnp.float32),
            pltpu.SemaphoreType.DMA,
            pltpu.SemaphoreType.DMA,
        ],
    )
    def k(data_hbm, idx_hbm, z_hbm, out_hbm, idx_v, dbuf0, dbuf1, accum,
          sem0, sem1):
        c_ax = lax.axis_index("c")
        s_ax = lax.axis_index("s")
        wid = s_ax * _NC + c_ax
        base = wid * per_w
        row0 = s_ax * rows_t
        pltpu.sync_copy(z_hbm.at[pl.ds(row0, rows_t)],
                        accum.at[pl.ds(row0, rows_t)])
        pltpu.sync_copy(idx_hbm.at[wid], idx_v)
        plsc.subcore_barrier()

        def step(i, carry):
            j0 = i * 2
            cp0 = pltpu.async_copy(
                data_hbm.at[pl.ds(base + j0 * _CH, _CH), pl.ds(c0, C)],
                dbuf0, sem0)
            cp1 = pltpu.async_copy(
                data_hbm.at[pl.ds(base + (j0 + 1) * _CH, _CH), pl.ds(c0, C)],
                dbuf1, sem1)
            cp0.wait()
            pltpu.sync_copy(dbuf0, accum.at[idx_v.at[j0]], add=True)
            cp1.wait()
            pltpu.sync_copy(dbuf1, accum.at[idx_v.at[j0 + 1]], add=True)
            return carry

        lax.fori_loop(0, nch // 2, step, 0)
        if nch % 2:
            j = nch - 1
            pltpu.async_copy(
                data_hbm.at[pl.ds(base + j * _CH, _CH), pl.ds(c0, C)],
                dbuf0, sem0).wait()
            pltpu.sync_copy(dbuf0, accum.at[idx_v.at[j]], add=True)
        plsc.subcore_barrier()
        pltpu.sync_copy(accum.at[pl.ds(row0, rows_t)],
                        out_hbm.at[c_ax, pl.ds(row0, rows_t)])

    return k(data, idx3, zeros_np)


def kernel(x, pos, edge_index, period_vec, batch, elem_embed, W_embed, b_embed,
           rbf_centers, W_msg, W_filter, W_self, W_attr, w_sh, b_conv,
           W_post0, b_post0, W_post1, b_post1, scale, shift):
    N = pos.shape[0]
    E = edge_index.shape[1]
    DA = elem_embed.shape[1]
    D = W_embed.shape[1]
    K = rbf_centers.shape[0]
    DH = W_post0.shape[1]
    T = W_post1.shape[1]

    NB = 1024
    NPAD = ((N + NB - 1) // NB) * NB
    EB = _blk(E)

    src = edge_index[0].astype(jnp.int32)
    dst = edge_index[1].astype(jnp.int32)
    xi_p = jnp.pad(x.reshape(-1, 1).astype(jnp.int32), ((0, NPAD - N), (0, 0)))
    batch_p = jnp.pad(batch.astype(jnp.int32).reshape(-1, 1),
                      ((0, NPAD - N), (0, 0)), constant_values=G_SEGMENTS)
    pos128 = jnp.pad(pos, ((0, NPAD - N), (0, 125)))
    per_w = E // _NW
    nch = per_w // _CH
    dst3 = dst.reshape(_NW, nch, _CH)
    nch2 = (E // _NS) // _CH
    src16 = src.reshape(_NS, 2, nch2 // 2, _CH)
    dst16 = dst.reshape(_NS, 2, nch2 // 2, _CH)
    ds4 = jnp.stack([dst16, src16])
    z128 = jnp.zeros((NPAD, D), jnp.float32)
    z8 = jnp.zeros((NPAD, 8), jnp.float32)

    NE_PAD = 128
    elem_pad = jnp.pad(elem_embed, ((0, NE_PAD - elem_embed.shape[0]), (0, 0)))
    centers = rbf_centers.reshape(1, K)
    b_embed2 = b_embed.reshape(1, D)
    b0_2 = b_post0.reshape(1, DH)
    b1_2 = b_post1.reshape(1, T)
    z1bar = scale[0][:, None] * W_post1.T
    W0T = W_post0.T
    WmT = [W_msg[l].T for l in range(3)]
    WfT = [W_filter[l].T for l in range(3)]
    WsT = [W_self[l].T for l in range(3)]

    x_attr, h0 = _t1_embed(xi_p, elem_pad, W_embed, b_embed2, NB)
    pg = _sc_gather2(pos128, dst, src)
    geom = _t_geom(pg, period_vec, w_sh, EB)

    hs_l = []
    pre_l = []
    h = h0
    for l in range(3):
        hs = _sc_gather(h, src)
        hs_l.append(hs)
        m = _t2_edge_fwd(hs, geom, W_msg[l], W_filter[l], centers, l, EB)
        agg2 = _sc_scatter_add(m, dst3, z128, 0, D, NPAD)
        pre, h = _t3_node(agg2, h, x_attr, W_self[l], W_attr[l],
                          b_conv[l].reshape(1, D), NB)
        pre_l.append(pre)

    z0, energies = _t4_head(h, batch_p, W_post0, b0_2, W_post1, b1_2,
                            scale, shift, NB)

    hbar = _t5_head_bwd(z0, z1bar, W0T, NB)
    gacc = jnp.zeros((E, 8), jnp.float32)
    hbar_halves = None
    for l in range(2, -1, -1):
        prebar, hbar_self = _t6_node_bwd(hbar, hbar_halves, pre_l[l], WsT[l],
                                         NB, need_self=(l > 0))
        mbar = _sc_gather(prebar, dst)
        outsc, gacc = _t7_edge_bwd(mbar, hs_l[l], geom, gacc, W_msg[l],
                                   W_filter[l], WmT[l], WfT[l],
                                   w_sh[l].reshape(1, 4), centers, l, EB,
                                   need_scatter=(l > 0))
        if l > 0:
            hbar = hbar_self
            hbar_halves = _sc_scatter_dual(outsc, src16, z128, (0, D), D, NPAD)

    evb = gacc
    pb = _sc_scatter_dual(evb, ds4, z128, (0, 0), D, NPAD)
    fb = _t9_forces(pb, NB)

    forces = jnp.stack([fb[:N, 0:3], fb[:N, 4:7]], axis=1)
    return (energies, forces)


# speedup vs baseline: 3.0761x; 1.0917x over previous
import functools

import jax
import jax.numpy as jnp
import numpy as np
from jax import lax
from jax.experimental import pallas as pl
from jax.experimental.pallas import tpu as pltpu
from jax.experimental.pallas import tpu_sc as plsc

_NC, _NS = 2, 16
_NW = _NC * _NS
_CH = 80

GAMMA = 10.0
EPS = 1e-12
G_SEGMENTS = 64


def _silu(x):
    return x * jax.nn.sigmoid(x)


def _dsilu(x):
    s = jax.nn.sigmoid(x)
    return s * (1.0 + x * (1.0 - s))


def _blk(E):
    for eb in (2000, 1600, 1000, 800, 500, 400, 200, 100, 8):
        if E % eb == 0:
            return eb
    return E


def _t1_embed(xi, elem_pad, W_embed, b_embed, NB):
    NPAD = xi.shape[0]
    DA = elem_pad.shape[1]
    D = W_embed.shape[1]
    NE = elem_pad.shape[0]

    def body(xi_ref, elem_ref, we_ref, be_ref, xa_ref, h0_ref):
        ids = xi_ref[...]
        cols = lax.broadcasted_iota(jnp.int32, (NB, NE), 1)
        oh = (cols == ids).astype(jnp.float32)
        xa = jnp.dot(oh, elem_ref[...], preferred_element_type=jnp.float32)
        xa_ref[...] = xa
        h0_ref[...] = jnp.dot(xa, we_ref[...], preferred_element_type=jnp.float32) + be_ref[...]

    return pl.pallas_call(
        body,
        grid=(NPAD // NB,),
        in_specs=[
            pl.BlockSpec((NB, 1), lambda i: (i, 0)),
            pl.BlockSpec((NE, DA), lambda i: (0, 0)),
            pl.BlockSpec((DA, D), lambda i: (0, 0)),
            pl.BlockSpec((1, D), lambda i: (0, 0)),
        ],
        out_specs=[
            pl.BlockSpec((NB, DA), lambda i: (i, 0)),
            pl.BlockSpec((NB, D), lambda i: (i, 0)),
        ],
        out_shape=[
            jax.ShapeDtypeStruct((NPAD, DA), jnp.float32),
            jax.ShapeDtypeStruct((NPAD, D), jnp.float32),
        ],
    )(xi, elem_pad, W_embed, b_embed)


def _t_geom(pd, ps, pv, w_sh, EB):
    E = pd.shape[0]

    def body(pd_ref, ps_ref, pv_ref, wsh_ref, out_ref):
        ev = pd_ref[...][:, 0:3] - ps_ref[...][:, 0:3] + pv_ref[...]
        r = jnp.sqrt(jnp.sum(ev * ev, axis=1, keepdims=True) + EPS)
        u = ev / r
        wsh = wsh_ref[...]
        s_all = jnp.dot(u, wsh[:, 1:4].T, preferred_element_type=jnp.float32) + wsh[:, 0][None, :]
        z = jnp.zeros((ev.shape[0], 1), jnp.float32)
        out_ref[...] = jnp.concatenate([s_all, r, u, z], axis=1)

    return pl.pallas_call(
        body,
        grid=(E // EB,),
        in_specs=[
            pl.BlockSpec((EB, 128), lambda i: (i, 0)),
            pl.BlockSpec((EB, 128), lambda i: (i, 0)),
            pl.BlockSpec((EB, 3), lambda i: (i, 0)),
            pl.BlockSpec((3, 4), lambda i: (0, 0)),
        ],
        out_specs=pl.BlockSpec((EB, 8), lambda i: (i, 0)),
        out_shape=jax.ShapeDtypeStruct((E, 8), jnp.float32),
    )(pd, ps, pv, w_sh)


def _t2_edge_fwd(hs, geom, Wm, Wf, centers, l, EB):
    E, D = hs.shape
    K = Wf.shape[0]

    def body(hs_ref, g_ref, wm_ref, wf_ref, c_ref, m_ref):
        geomb = g_ref[...]
        r = geomb[:, 3:4]
        s = geomb[:, l:l + 1]
        rbf = jnp.exp(-GAMMA * (r - c_ref[...]) ** 2)
        filt = jnp.dot(rbf, wf_ref[...], preferred_element_type=jnp.float32)
        gg = jnp.dot(hs_ref[...], wm_ref[...], preferred_element_type=jnp.float32)
        m_ref[...] = gg * filt * s

    return pl.pallas_call(
        body,
        grid=(E // EB,),
        in_specs=[
            pl.BlockSpec((EB, D), lambda i: (i, 0)),
            pl.BlockSpec((EB, 8), lambda i: (i, 0)),
            pl.BlockSpec((D, D), lambda i: (0, 0)),
            pl.BlockSpec((K, D), lambda i: (0, 0)),
            pl.BlockSpec((1, K), lambda i: (0, 0)),
        ],
        out_specs=pl.BlockSpec((EB, D), lambda i: (i, 0)),
        out_shape=jax.ShapeDtypeStruct((E, D), jnp.float32),
    )(hs, geom, Wm, Wf, centers)


def _t3_node(agg2, h, x_attr, Ws, Wa, bc, NB):
    NPAD, D = h.shape
    DA = x_attr.shape[1]

    def body(a_ref, h_ref, xa_ref, ws_ref, wa_ref, bc_ref, pre_ref, hn_ref):
        pre = (a_ref[0] + a_ref[1]
               + jnp.dot(h_ref[...], ws_ref[...], preferred_element_type=jnp.float32)
               + jnp.dot(xa_ref[...], wa_ref[...], preferred_element_type=jnp.float32)
               + bc_ref[...])
        pre_ref[...] = pre
        hn_ref[...] = _silu(pre)

    return pl.pallas_call(
        body,
        grid=(NPAD // NB,),
        in_specs=[
            pl.BlockSpec((2, NB, D), lambda i: (0, i, 0)),
            pl.BlockSpec((NB, D), lambda i: (i, 0)),
            pl.BlockSpec((NB, DA), lambda i: (i, 0)),
            pl.BlockSpec((D, D), lambda i: (0, 0)),
            pl.BlockSpec((DA, D), lambda i: (0, 0)),
            pl.BlockSpec((1, D), lambda i: (0, 0)),
        ],
        out_specs=[
            pl.BlockSpec((NB, D), lambda i: (i, 0)),
            pl.BlockSpec((NB, D), lambda i: (i, 0)),
        ],
        out_shape=[
            jax.ShapeDtypeStruct((NPAD, D), jnp.float32),
            jax.ShapeDtypeStruct((NPAD, D), jnp.float32),
        ],
    )(agg2, h, x_attr, Ws, Wa, bc)


def _t4_head(h, batch_p, W0, b0, W1, b1, scale, shift, NB):
    NPAD, D = h.shape
    DH = W0.shape[1]
    T = W1.shape[1]
    nblocks = NPAD // NB

    def body(h_ref, b_ref, w0_ref, b0_ref, w1_ref, b1_ref, sc_ref, sh_ref,
             z0_ref, en_ref):
        i = pl.program_id(0)
        z0 = jnp.dot(h_ref[...], w0_ref[...], preferred_element_type=jnp.float32) + b0_ref[...]
        z0_ref[...] = z0
        o = jnp.dot(_silu(z0), w1_ref[...], preferred_element_type=jnp.float32) + b1_ref[...]
        seg = b_ref[...]
        rows = lax.broadcasted_iota(jnp.int32, (G_SEGMENTS, NB), 0)
        oh = (rows == seg[:, 0][None, :]).astype(jnp.float32)
        part = jnp.dot(oh, o, preferred_element_type=jnp.float32)

        @pl.when(i == 0)
        def _():
            en_ref[...] = jnp.zeros_like(en_ref)

        en_ref[...] += part

        @pl.when(i == nblocks - 1)
        def _():
            en_ref[...] = en_ref[...] * sc_ref[...] + sh_ref[...]

    return pl.pallas_call(
        body,
        grid=(nblocks,),
        in_specs=[
            pl.BlockSpec((NB, D), lambda i: (i, 0)),
            pl.BlockSpec((NB, 1), lambda i: (i, 0)),
            pl.BlockSpec((D, DH), lambda i: (0, 0)),
            pl.BlockSpec((1, DH), lambda i: (0, 0)),
            pl.BlockSpec((DH, T), lambda i: (0, 0)),
            pl.BlockSpec((1, T), lambda i: (0, 0)),
            pl.BlockSpec((1, T), lambda i: (0, 0)),
            pl.BlockSpec((1, T), lambda i: (0, 0)),
        ],
        out_specs=[
            pl.BlockSpec((NB, DH), lambda i: (i, 0)),
            pl.BlockSpec((G_SEGMENTS, T), lambda i: (0, 0)),
        ],
        out_shape=[
            jax.ShapeDtypeStruct((NPAD, DH), jnp.float32),
            jax.ShapeDtypeStruct((G_SEGMENTS, T), jnp.float32),
        ],
    )(h, batch_p, W0, b0, W1, b1, scale, shift)


def _t5_head_bwd(z0, z1bar, W0T, NB):
    NPAD, DH = z0.shape
    D = W0T.shape[1]

    def body(z0_ref, zb_ref, w0t_ref, hb_ref):
        d = _dsilu(z0_ref[...])
        zb = zb_ref[...]
        h0 = jnp.dot(d * zb[0][None, :], w0t_ref[...], preferred_element_type=jnp.float32)
        h1 = jnp.dot(d * zb[1][None, :], w0t_ref[...], preferred_element_type=jnp.float32)
        hb_ref[...] = jnp.concatenate([h0, h1], axis=1)

    return pl.pallas_call(
        body,
        grid=(NPAD // NB,),
        in_specs=[
            pl.BlockSpec((NB, DH), lambda i: (i, 0)),
            pl.BlockSpec((2, DH), lambda i: (0, 0)),
            pl.BlockSpec((DH, D), lambda i: (0, 0)),
        ],
        out_specs=pl.BlockSpec((NB, 2 * D), lambda i: (i, 0)),
        out_shape=jax.ShapeDtypeStruct((NPAD, 2 * D), jnp.float32),
    )(z0, z1bar, W0T)


def _t6_node_bwd(hbar_full, hbar_halves, pre, WsT, NB, need_self):
    NPAD, D = pre.shape
    have_halves = hbar_halves is not None

    def body(*refs):
        i = 0
        hf_ref = refs[i]; i += 1
        if have_halves:
            ha_ref = refs[i]; i += 1
        pre_ref = refs[i]; wst_ref = refs[i + 1]; i += 2
        pb_ref = refs[i]; i += 1
        acc = hf_ref[...]
        if have_halves:
            acc = acc + jnp.concatenate([ha_ref[0], ha_ref[1]], axis=1)
        d = _dsilu(pre_ref[...])
        prebar = acc * jnp.concatenate([d, d], axis=1)
        pb_ref[...] = prebar
        if need_self:
            hs_ref = refs[i]
            wst = wst_ref[...]
            h0 = jnp.dot(prebar[:, :D], wst, preferred_element_type=jnp.float32)
            h1 = jnp.dot(prebar[:, D:], wst, preferred_element_type=jnp.float32)
            hs_ref[...] = jnp.concatenate([h0, h1], axis=1)

    in_specs = [pl.BlockSpec((NB, 2 * D), lambda i: (i, 0))]
    ins = [hbar_full]
    if have_halves:
        in_specs += [pl.BlockSpec((2, NB, D), lambda i: (0, i, 0))]
        ins += [hbar_halves]
    in_specs += [
        pl.BlockSpec((NB, D), lambda i: (i, 0)),
        pl.BlockSpec((D, D), lambda i: (0, 0)),
    ]
    ins += [pre, WsT]

    out_specs = [pl.BlockSpec((NB, 2 * D), lambda i: (i, 0))]
    out_shape = [jax.ShapeDtypeStruct((NPAD, 2 * D), jnp.float32)]
    if need_self:
        out_specs.append(pl.BlockSpec((NB, 2 * D), lambda i: (i, 0)))
        out_shape.append(jax.ShapeDtypeStruct((NPAD, 2 * D), jnp.float32))

    res = pl.pallas_call(
        body,
        grid=(NPAD // NB,),
        in_specs=in_specs,
        out_specs=out_specs,
        out_shape=out_shape,
    )(*ins)
    return res if need_self else (res[0], None)


def _t7_edge_bwd(mbar, hs, geom, gacc, Wm, Wf, WmT, WfT, wsh_l, centers, l, EB,
                 need_scatter):
    E, D = hs.shape
    K = Wf.shape[0]

    def body(mb_ref, hs_ref, g_ref, ga_ref, wm_ref, wf_ref, wmt_ref, wft_ref,
             wshl_ref, c_ref, *outs):
        geomb = g_ref[...]
        r = geomb[:, 3:4]
        s = geomb[:, l:l + 1]
        cen = c_ref[...]
        rbf = jnp.exp(-GAMMA * (r - cen) ** 2)
        drbf = (-2.0 * GAMMA) * (r - cen) * rbf
        filt = jnp.dot(rbf, wf_ref[...], preferred_element_type=jnp.float32)
        g = jnp.dot(hs_ref[...], wm_ref[...], preferred_element_type=jnp.float32)
        mb = mb_ref[...]
        wshl = wshl_ref[...]
        fs = filt * s
        r1 = lax.broadcasted_iota(jnp.int32, (1, 8), 1)
        r4 = lax.broadcasted_iota(jnp.int32, (4, 8), 0)
        c4 = lax.broadcasted_iota(jnp.int32, (4, 8), 1)
        E_row = [(r1 == 4 * i).astype(jnp.float32) for i in range(2)]
        P_row = [((c4 == 4 * i + r4) & (r4 >= 1)).astype(jnp.float32)
                 for i in range(2)]
        ga_total = ga_ref[...]
        sc_parts = []
        for i in range(2):
            mbi = mb[:, i * D:(i + 1) * D]
            if need_scatter:
                gbar = mbi * fs
                sc_parts.append(jnp.dot(gbar, wmt_ref[...], preferred_element_type=jnp.float32))
            mg = mbi * g
            fbar = jnp.dot(mg * s, wft_ref[...], preferred_element_type=jnp.float32)
            rbar = jnp.sum(fbar * drbf, axis=1, keepdims=True)
            sb = jnp.sum(mg * filt, axis=1, keepdims=True)
            ga_total = (ga_total
                        + jnp.dot(rbar, E_row[i],
                                  preferred_element_type=jnp.float32)
                        + jnp.dot(jnp.dot(sb, wshl,
                                          preferred_element_type=jnp.float32),
                                  P_row[i],
                                  preferred_element_type=jnp.float32))
        ga_out = outs[-1]
        if need_scatter:
            ga_out[...] = ga_total
            outs[0][...] = jnp.concatenate(sc_parts, axis=1)
        else:
            r8r = lax.broadcasted_iota(jnp.int32, (8, 8), 0)
            c8 = lax.broadcasted_iota(jnp.int32, (8, 8), 1)
            M_U = ((r8r >= 4) & (r8r <= 6)
                   & ((c8 == r8r - 3) | (c8 == r8r + 1))).astype(jnp.float32)
            M_R = (r8r == 3).astype(jnp.float32)
            M_G = ((r8r // 4) == (c8 // 4)).astype(jnp.float32)
            M_RB = (((r8r == 0) & (c8 < 4))
                    | ((r8r == 4) & (c8 >= 4))).astype(jnp.float32)
            mask_ub = (r1 % 4 != 0).astype(jnp.float32)
            u8 = jnp.dot(geomb, M_U, preferred_element_type=jnp.float32)
            r8 = jnp.dot(geomb, M_R, preferred_element_type=jnp.float32)
            uu8 = jnp.dot(ga_total * u8, M_G, preferred_element_type=jnp.float32)
            rb8 = jnp.dot(ga_total, M_RB, preferred_element_type=jnp.float32)
            evb8 = (ga_total * mask_ub - u8 * uu8) / r8 + u8 * rb8
            ztail = jnp.zeros((geomb.shape[0], 120), jnp.float32)
            ga_out[...] = jnp.concatenate([evb8, ztail], axis=1)

    out_specs = []
    out_shape = []
    if need_scatter:
        out_specs.append(pl.BlockSpec((EB, 2 * D), lambda i: (i, 0)))
        out_shape.append(jax.ShapeDtypeStruct((E, 2 * D), jnp.float32))
        out_specs.append(pl.BlockSpec((EB, 8), lambda i: (i, 0)))
        out_shape.append(jax.ShapeDtypeStruct((E, 8), jnp.float32))
    else:
        out_specs.append(pl.BlockSpec((EB, 128), lambda i: (i, 0)))
        out_shape.append(jax.ShapeDtypeStruct((E, 128), jnp.float32))

    res = pl.pallas_call(
        body,
        grid=(E // EB,),
        in_specs=[
            pl.BlockSpec((EB, 2 * D), lambda i: (i, 0)),
            pl.BlockSpec((EB, D), lambda i: (i, 0)),
            pl.BlockSpec((EB, 8), lambda i: (i, 0)),
            pl.BlockSpec((EB, 8), lambda i: (i, 0)),
            pl.BlockSpec((D, D), lambda i: (0, 0)),
            pl.BlockSpec((K, D), lambda i: (0, 0)),
            pl.BlockSpec((D, D), lambda i: (0, 0)),
            pl.BlockSpec((D, K), lambda i: (0, 0)),
            pl.BlockSpec((1, 4), lambda i: (0, 0)),
            pl.BlockSpec((1, K), lambda i: (0, 0)),
        ],
        out_specs=out_specs,
        out_shape=out_shape,
    )(mbar, hs, geom, gacc, Wm, Wf, WmT, WfT, wsh_l, centers)
    if need_scatter:
        return res[0], res[1]
    return None, res[0]


def _t9_forces(pb, NB):
    NPAD = pb.shape[1]

    def body(pb_ref, out_ref):
        f = pb_ref[1] - pb_ref[0]
        out_ref[...] = f[:, 0:8]

    return pl.pallas_call(
        body,
        grid=(NPAD // NB,),
        in_specs=[pl.BlockSpec((2, NB, 128), lambda i: (0, i, 0))],
        out_specs=pl.BlockSpec((NB, 8), lambda i: (i, 0)),
        out_shape=jax.ShapeDtypeStruct((NPAD, 8), jnp.float32),
    )(pb)


def _sc_gather(table, idx):
    C = table.shape[1]
    E = idx.shape[0]
    per_w = E // _NW
    nch = per_w // _CH
    npair = nch // 2
    mesh = plsc.VectorSubcoreMesh(core_axis_name="c", subcore_axis_name="s")

    @functools.partial(
        pl.kernel,
        out_type=jax.ShapeDtypeStruct((E, C), jnp.float32),
        mesh=mesh,
        scratch_types=[
            pltpu.VMEM((per_w,), jnp.int32),
            pltpu.VMEM((_CH, C), jnp.float32),
            pltpu.VMEM((_CH, C), jnp.float32),
            pltpu.SemaphoreType.DMA,
            pltpu.SemaphoreType.DMA,
        ],
    )
    def k(table_hbm, idx_hbm, out_hbm, idx_v, buf0, buf1, sem0, sem1):
        wid = lax.axis_index("s") * _NC + lax.axis_index("c")
        base = wid * per_w
        pltpu.sync_copy(idx_hbm.at[pl.ds(base, per_w)], idx_v)

        def step(i, carry):
            r0 = i * (2 * _CH)
            cp0 = pltpu.async_copy(table_hbm.at[idx_v.at[pl.ds(r0, _CH)]], buf0, sem0)
            cp1 = pltpu.async_copy(
                table_hbm.at[idx_v.at[pl.ds(r0 + _CH, _CH)]], buf1, sem1)
            cp0.wait()
            pltpu.sync_copy(buf0, out_hbm.at[pl.ds(base + r0, _CH)])
            cp1.wait()
            pltpu.sync_copy(buf1, out_hbm.at[pl.ds(base + r0 + _CH, _CH)])
            return carry

        lax.fori_loop(0, npair, step, 0)
        if nch % 2:
            r0 = (nch - 1) * _CH
            pltpu.async_copy(table_hbm.at[idx_v.at[pl.ds(r0, _CH)]], buf0, sem0).wait()
            pltpu.sync_copy(buf0, out_hbm.at[pl.ds(base + r0, _CH)])

    return k(table, idx)


def _sc_gather2(table, idx_a, idx_b):
    C = table.shape[1]
    E = idx_a.shape[0]
    per_w = E // _NW
    nch = per_w // _CH
    mesh = plsc.VectorSubcoreMesh(core_axis_name="c", subcore_axis_name="s")

    @functools.partial(
        pl.kernel,
        out_type=(jax.ShapeDtypeStruct((E, C), jnp.float32),
                  jax.ShapeDtypeStruct((E, C), jnp.float32)),
        mesh=mesh,
        scratch_types=[
            pltpu.VMEM((per_w,), jnp.int32),
            pltpu.VMEM((per_w,), jnp.int32),
            pltpu.VMEM((_CH, C), jnp.float32),
            pltpu.VMEM((_CH, C), jnp.float32),
            pltpu.SemaphoreType.DMA,
            pltpu.SemaphoreType.DMA,
        ],
    )
    def k(table_hbm, ia_hbm, ib_hbm, outa_hbm, outb_hbm, ia_v, ib_v,
          buf0, buf1, sem0, sem1):
        wid = lax.axis_index("s") * _NC + lax.axis_index("c")
        base = wid * per_w
        pltpu.sync_copy(ia_hbm.at[pl.ds(base, per_w)], ia_v)
        pltpu.sync_copy(ib_hbm.at[pl.ds(base, per_w)], ib_v)

        def step(i, carry):
            r0 = i * _CH
            cp0 = pltpu.async_copy(table_hbm.at[ia_v.at[pl.ds(r0, _CH)]], buf0, sem0)
            cp1 = pltpu.async_copy(table_hbm.at[ib_v.at[pl.ds(r0, _CH)]], buf1, sem1)
            cp0.wait()
            pltpu.sync_copy(buf0, outa_hbm.at[pl.ds(base + r0, _CH)])
            cp1.wait()
            pltpu.sync_copy(buf1, outb_hbm.at[pl.ds(base + r0, _CH)])
            return carry

        lax.fori_loop(0, nch, step, 0)

    return k(table, idx_a, idx_b)


def _sc_scatter_dual(data, idxs, zeros_np, c0_pair, C, npad):
    E = data.shape[0]
    per_w = E // _NS
    nch = per_w // _CH
    NPH = 2
    nph_ch = nch // NPH
    rows_t = npad // _NS
    c0a, c0b = c0_pair
    idx_per_core = idxs.ndim == 5
    mesh = plsc.VectorSubcoreMesh(core_axis_name="c", subcore_axis_name="s")

    @functools.partial(
        pl.kernel,
        out_type=jax.ShapeDtypeStruct((2, npad, C), jnp.float32),
        mesh=mesh,
        scratch_types=[
            pltpu.VMEM((nph_ch, _CH), jnp.int32),
            pltpu.VMEM((_CH, C), jnp.float32),
            pltpu.VMEM((_CH, C), jnp.float32),
            pltpu.VMEM_SHARED((npad, C), jnp.float32),
            pltpu.SemaphoreType.DMA,
            pltpu.SemaphoreType.DMA,
        ],
    )
    def k(data_hbm, idx_hbm, z_hbm, out_hbm, idx_v, dbuf0, dbuf1, accum,
          sem0, sem1):
        c_ax = lax.axis_index("c")
        s_ax = lax.axis_index("s")
        base = s_ax * per_w
        row0 = s_ax * rows_t
        if c0a == c0b:
            c0 = c0a
        else:
            c0 = c0a + c_ax * (c0b - c0a)
        pltpu.sync_copy(z_hbm.at[pl.ds(row0, rows_t)],
                        accum.at[pl.ds(row0, rows_t)])
        plsc.subcore_barrier()
        for ph in range(NPH):
            if idx_per_core:
                pltpu.sync_copy(idx_hbm.at[c_ax, s_ax, ph], idx_v)
            else:
                pltpu.sync_copy(idx_hbm.at[s_ax, ph], idx_v)
            pbase = base + ph * nph_ch * _CH

            def step(i, carry):
                j0 = i * 2
                cp0 = pltpu.async_copy(
                    data_hbm.at[pl.ds(pbase + j0 * _CH, _CH), pl.ds(c0, C)],
                    dbuf0, sem0)
                cp1 = pltpu.async_copy(
                    data_hbm.at[pl.ds(pbase + (j0 + 1) * _CH, _CH),
                                pl.ds(c0, C)],
                    dbuf1, sem1)
                cp0.wait()
                pltpu.sync_copy(dbuf0, accum.at[idx_v.at[j0]], add=True)
                cp1.wait()
                pltpu.sync_copy(dbuf1, accum.at[idx_v.at[j0 + 1]], add=True)
                return carry

            lax.fori_loop(0, nph_ch // 2, step, 0)
            if nph_ch % 2:
                j = nph_ch - 1
                pltpu.async_copy(
                    data_hbm.at[pl.ds(pbase + j * _CH, _CH), pl.ds(c0, C)],
                    dbuf0, sem0).wait()
                pltpu.sync_copy(dbuf0, accum.at[idx_v.at[j]], add=True)
        plsc.subcore_barrier()
        pltpu.sync_copy(accum.at[pl.ds(row0, rows_t)],
                        out_hbm.at[c_ax, pl.ds(row0, rows_t)])

    return k(data, idxs, zeros_np)


def _sc_scatter_add(data, idx3, zeros_np, c0, C, npad):
    E = data.shape[0]
    per_w = E // _NW
    nch = per_w // _CH
    rows_t = npad // _NS
    mesh = plsc.VectorSubcoreMesh(core_axis_name="c", subcore_axis_name="s")

    @functools.partial(
        pl.kernel,
        out_type=jax.ShapeDtypeStruct((2, npad, C), jnp.float32),
        mesh=mesh,
        scratch_types=[
            pltpu.VMEM((nch, _CH), jnp.int32),
            pltpu.VMEM((_CH, C), jnp.float32),
            pltpu.VMEM((_CH, C), jnp.float32),
            pltpu.VMEM_SHARED((npad, C), jnp.float32),
            pltpu.SemaphoreType.DMA,
            pltpu.SemaphoreType.DMA,
        ],
    )
    def k(data_hbm, idx_hbm, z_hbm, out_hbm, idx_v, dbuf0, dbuf1, accum,
          sem0, sem1):
        c_ax = lax.axis_index("c")
        s_ax = lax.axis_index("s")
        wid = s_ax * _NC + c_ax
        base = wid * per_w
        row0 = s_ax * rows_t
        pltpu.sync_copy(z_hbm.at[pl.ds(row0, rows_t)],
                        accum.at[pl.ds(row0, rows_t)])
        pltpu.sync_copy(idx_hbm.at[wid], idx_v)
        plsc.subcore_barrier()

        def step(i, carry):
            j0 = i * 2
            cp0 = pltpu.async_copy(
                data_hbm.at[pl.ds(base + j0 * _CH, _CH), pl.ds(c0, C)],
                dbuf0, sem0)
            cp1 = pltpu.async_copy(
                data_hbm.at[pl.ds(base + (j0 + 1) * _CH, _CH), pl.ds(c0, C)],
                dbuf1, sem1)
            cp0.wait()
            pltpu.sync_copy(dbuf0, accum.at[idx_v.at[j0]], add=True)
            cp1.wait()
            pltpu.sync_copy(dbuf1, accum.at[idx_v.at[j0 + 1]], add=True)
            return carry

        lax.fori_loop(0, nch // 2, step, 0)
        if nch % 2:
            j = nch - 1
            pltpu.async_copy(
                data_hbm.at[pl.ds(base + j * _CH, _CH), pl.ds(c0, C)],
                dbuf0, sem0).wait()
            pltpu.sync_copy(dbuf0, accum.at[idx_v.at[j]], add=True)
        plsc.subcore_barrier()
        pltpu.sync_copy(accum.at[pl.ds(row0, rows_t)],
                        out_hbm.at[c_ax, pl.ds(row0, rows_t)])

    return k(data, idx3, zeros_np)


def kernel(x, pos, edge_index, period_vec, batch, elem_embed, W_embed, b_embed,
           rbf_centers, W_msg, W_filter, W_self, W_attr, w_sh, b_conv,
           W_post0, b_post0, W_post1, b_post1, scale, shift):
    N = pos.shape[0]
    E = edge_index.shape[1]
    DA = elem_embed.shape[1]
    D = W_embed.shape[1]
    K = rbf_centers.shape[0]
    DH = W_post0.shape[1]
    T = W_post1.shape[1]

    NB = 1024
    NPAD = ((N + NB - 1) // NB) * NB
    EB = _blk(E)

    src = edge_index[0].astype(jnp.int32)
    dst = edge_index[1].astype(jnp.int32)
    xi_p = jnp.pad(x.reshape(-1, 1).astype(jnp.int32), ((0, NPAD - N), (0, 0)))
    batch_p = jnp.pad(batch.astype(jnp.int32).reshape(-1, 1),
                      ((0, NPAD - N), (0, 0)), constant_values=G_SEGMENTS)
    pos128 = jnp.pad(pos, ((0, NPAD - N), (0, 125)))
    per_w = E // _NW
    nch = per_w // _CH
    dst3 = dst.reshape(_NW, nch, _CH)
    nch2 = (E // _NS) // _CH
    src16 = src.reshape(_NS, 2, nch2 // 2, _CH)
    dst16 = dst.reshape(_NS, 2, nch2 // 2, _CH)
    ds4 = jnp.stack([dst16, src16])
    z128 = jnp.zeros((NPAD, D), jnp.float32)
    z8 = jnp.zeros((NPAD, 8), jnp.float32)

    NE_PAD = 128
    elem_pad = jnp.pad(elem_embed, ((0, NE_PAD - elem_embed.shape[0]), (0, 0)))
    centers = rbf_centers.reshape(1, K)
    b_embed2 = b_embed.reshape(1, D)
    b0_2 = b_post0.reshape(1, DH)
    b1_2 = b_post1.reshape(1, T)
    z1bar = scale[0][:, None] * W_post1.T
    W0T = W_post0.T
    WmT = [W_msg[l].T for l in range(3)]
    WfT = [W_filter[l].T for l in range(3)]
    WsT = [W_self[l].T for l in range(3)]

    x_attr, h0 = _t1_embed(xi_p, elem_pad, W_embed, b_embed2, NB)
    pd_rows, ps_rows = _sc_gather2(pos128, dst, src)
    geom = _t_geom(pd_rows, ps_rows, period_vec, w_sh, EB)

    hs_l = []
    pre_l = []
    h = h0
    for l in range(3):
        hs = _sc_gather(h, src)
        hs_l.append(hs)
        m = _t2_edge_fwd(hs, geom, W_msg[l], W_filter[l], centers, l, EB)
        agg2 = _sc_scatter_add(m, dst3, z128, 0, D, NPAD)
        pre, h = _t3_node(agg2, h, x_attr, W_self[l], W_attr[l],
                          b_conv[l].reshape(1, D), NB)
        pre_l.append(pre)

    z0, energies = _t4_head(h, batch_p, W_post0, b0_2, W_post1, b1_2,
                            scale, shift, NB)

    hbar = _t5_head_bwd(z0, z1bar, W0T, NB)
    gacc = jnp.zeros((E, 8), jnp.float32)
    hbar_halves = None
    for l in range(2, -1, -1):
        prebar, hbar_self = _t6_node_bwd(hbar, hbar_halves, pre_l[l], WsT[l],
                                         NB, need_self=(l > 0))
        mbar = _sc_gather(prebar, dst)
        outsc, gacc = _t7_edge_bwd(mbar, hs_l[l], geom, gacc, W_msg[l],
                                   W_filter[l], WmT[l], WfT[l],
                                   w_sh[l].reshape(1, 4), centers, l, EB,
                                   need_scatter=(l > 0))
        if l > 0:
            hbar = hbar_self
            hbar_halves = _sc_scatter_dual(outsc, src16, z128, (0, D), D, NPAD)

    evb = gacc
    pb = _sc_scatter_dual(evb, ds4, z128, (0, 0), D, NPAD)
    fb = _t9_forces(pb, NB)

    forces = jnp.stack([fb[:N, 1:4], fb[:N, 5:8]], axis=1)
    return (energies, forces)
